# Initial kernel scaffold; baseline (speedup 1.0000x reference)
import jax
import jax.numpy as jnp
from jax.experimental import pallas as pl


def kernel(obs, actions, q1_W0, q1_b0, q1_W1, q1_b1, q1_W2, q1_b2, q1_W3, q1_b3, q2_W0, q2_b0, q2_W1, q2_b1, q2_W2, q2_b2, q2_W3, q2_b3):
    raise NotImplementedError("write your pallas kernel here")


# speedup vs baseline: 2.1221x; 2.1221x over previous
import functools

import jax
import jax.numpy as jnp
from jax import lax
from jax.experimental import pallas as pl
from jax.experimental.pallas import tpu as pltpu
from jax.experimental.pallas import tpu_sc as plsc

B = 4096
OBS_DIM = 1024
ACT_DIM = 32
N_TASKS = 8
HID = 1024
BLK = 256
G = B // BLK + N_TASKS
PAD = G * BLK

ACT_PAD = 128
NC = 2
NS = 16
NW = NC * NS
RPW = B // NW
CH = 32
NCH = RPW // CH


def _route_body(obs8_ref, pos_ref, be_ref, nv_ref):
    v = obs8_ref[...]
    best = v[0:1]
    idx = jnp.zeros((1, B), jnp.int32)
    for t in range(1, N_TASKS):
        m = v[t:t + 1] > best
        idx = jnp.where(m, t, idx)
        best = jnp.where(m, v[t:t + 1], best)
    tid = lax.broadcasted_iota(jnp.int32, (N_TASKS, B), 0)
    oh = (idx == tid).astype(jnp.float32)
    cs = oh
    s = 1
    while s < B:
        cs = cs + jnp.concatenate(
            [jnp.zeros((N_TASKS, s), jnp.float32), cs[:, :B - s]], axis=1)
        s *= 2
    counts = cs[:, B - 1:B]
    nb = jnp.floor((counts + (BLK - 1)) / BLK)
    inc = nb
    s = 1
    while s < N_TASKS:
        inc = inc + jnp.concatenate(
            [jnp.zeros((s, 1), jnp.float32), inc[:N_TASKS - s]], axis=0)
        s *= 2
    base = (inc - nb) * BLK
    posf = jnp.sum(oh * (base + cs - 1.0), axis=0, keepdims=True)
    pos_ref[...] = posf.astype(jnp.int32)
    bi = lax.broadcasted_iota(jnp.int32, (N_TASKS, G), 1)
    inc_i = inc.astype(jnp.int32)
    nb_i = nb.astype(jnp.int32)
    be = jnp.sum((bi >= inc_i).astype(jnp.int32), axis=0, keepdims=True)
    be_ref[...] = jnp.minimum(be, N_TASKS - 1)
    first = inc_i - nb_i
    owner = jnp.logical_and(bi >= first, bi < inc_i)
    vraw = counts.astype(jnp.int32) - (bi - first) * BLK
    vclamp = jnp.clip(vraw, 0, BLK)
    nv_ref[...] = jnp.sum(jnp.where(owner, vclamp, 0), axis=0, keepdims=True)


def _route(obs8):
    return pl.pallas_call(
        _route_body,
        out_shape=[jax.ShapeDtypeStruct((1, B), jnp.int32),
                   jax.ShapeDtypeStruct((1, G), jnp.int32),
                   jax.ShapeDtypeStruct((1, G), jnp.int32)],
    )(obs8)


def _scatter_body(obs_hbm, act_hbm, pos_hbm, xso_hbm, xsa_hbm,
                  idx_v, obuf, abuf, sem):
    wid = lax.axis_index("s") * NC + lax.axis_index("c")
    base = wid * RPW
    pltpu.sync_copy(pos_hbm.at[wid], idx_v)
    for ch in range(NCH):
        pltpu.sync_copy(obs_hbm.at[pl.ds(base + ch * CH, CH)], obuf)
        pltpu.async_copy(obuf, xso_hbm.at[idx_v.at[ch]], sem).wait()
        pltpu.sync_copy(act_hbm.at[pl.ds(base + ch * CH, CH)], abuf)
        pltpu.async_copy(abuf, xsa_hbm.at[idx_v.at[ch]], sem).wait()


def _scatter(obs, actions, pos3):
    return pl.kernel(
        _scatter_body,
        out_type=[jax.ShapeDtypeStruct((PAD, OBS_DIM), jnp.float32),
                  jax.ShapeDtypeStruct((PAD, ACT_PAD), jnp.float32)],
        mesh=plsc.VectorSubcoreMesh(core_axis_name="c", subcore_axis_name="s"),
        scratch_types=[
            pltpu.VMEM((NCH, CH), jnp.int32),
            pltpu.VMEM((CH, OBS_DIM), jnp.float32),
            pltpu.VMEM((CH, ACT_PAD), jnp.float32),
            pltpu.SemaphoreType.DMA,
        ],
    )(obs, actions, pos3)


def _mlp_body(be_ref, nv_ref, xo_ref, xa_ref,
              a_w0a, a_w0b, a_b0, a_w1, a_b1, a_w2, a_b2, a_w3, a_b3,
              c_w0a, c_w0b, c_b0, c_w1, c_b1, c_w2, c_b2, c_w3, c_b3,
              q1o_ref, q2o_ref):
    del be_ref
    valid = nv_ref[pl.program_id(0)]
    rmask = lax.broadcasted_iota(jnp.int32, (BLK, 1), 0) < valid
    xo = xo_ref[...]
    xa = xa_ref[...][:, :ACT_DIM]

    def head(w0a, w0b, b0, w1, b1, w2, b2, w3, b3):
        h = (jnp.dot(xo, w0a[0], preferred_element_type=jnp.float32)
             + jnp.dot(xa, w0b[0], preferred_element_type=jnp.float32)
             + b0[0])
        h = jnp.maximum(h, 0.0)
        h = jnp.maximum(
            jnp.dot(h, w1[0], preferred_element_type=jnp.float32) + b1[0], 0.0)
        h = jnp.maximum(
            jnp.dot(h, w2[0], preferred_element_type=jnp.float32) + b2[0], 0.0)
        return jnp.dot(h, w3[0], preferred_element_type=jnp.float32) + b3[0]

    q1o_ref[...] = jnp.where(
        rmask, head(a_w0a, a_w0b, a_b0, a_w1, a_b1, a_w2, a_b2, a_w3, a_b3), 0.0)
    q2o_ref[...] = jnp.where(
        rmask, head(c_w0a, c_w0b, c_b0, c_w1, c_b1, c_w2, c_b2, c_w3, c_b3), 0.0)


def _grouped_mlp(be, nv, xso, xsa, wts):
    def xmap(b, eb, nv_):
        return (b, 0)

    def emap3(b, eb, nv_):
        return (eb[b], 0, 0)

    w_specs = []
    for _ in range(2):
        w_specs += [
            pl.BlockSpec((1, OBS_DIM, HID), emap3),
            pl.BlockSpec((1, ACT_DIM, HID), emap3),
            pl.BlockSpec((1, 1, HID), emap3),
            pl.BlockSpec((1, HID, HID), emap3),
            pl.BlockSpec((1, 1, HID), emap3),
            pl.BlockSpec((1, HID, HID), emap3),
            pl.BlockSpec((1, 1, HID), emap3),
            pl.BlockSpec((1, HID, 1), emap3),
            pl.BlockSpec((1, 1, 1), emap3),
        ]
    grid_spec = pltpu.PrefetchScalarGridSpec(
        num_scalar_prefetch=2,
        grid=(G,),
        in_specs=[pl.BlockSpec((BLK, OBS_DIM), xmap),
                  pl.BlockSpec((BLK, ACT_PAD), xmap)] + w_specs,
        out_specs=[pl.BlockSpec((BLK, 1), xmap),
                   pl.BlockSpec((BLK, 1), xmap)],
    )
    return pl.pallas_call(
        _mlp_body,
        grid_spec=grid_spec,
        out_shape=[jax.ShapeDtypeStruct((PAD, 1), jnp.float32),
                   jax.ShapeDtypeStruct((PAD, 1), jnp.float32)],
    )(be, nv, xso, xsa, *wts)


UBLK = 256


def _unpermute_body(pos_ref, q1p_ref, q2p_ref, q1_ref, q2_ref):
    p = pos_ref[...]
    sel = (p == lax.broadcasted_iota(jnp.int32, (UBLK, PAD), 1)
           ).astype(jnp.float32)
    q1_ref[...] = jnp.dot(sel, q1p_ref[...],
                          preferred_element_type=jnp.float32)
    q2_ref[...] = jnp.dot(sel, q2p_ref[...],
                          preferred_element_type=jnp.float32)


def _unpermute(pos_col, q1p, q2p):
    return pl.pallas_call(
        _unpermute_body,
        grid=(B // UBLK,),
        in_specs=[pl.BlockSpec((UBLK, 1), lambda b: (b, 0)),
                  pl.BlockSpec((PAD, 1), lambda b: (0, 0)),
                  pl.BlockSpec((PAD, 1), lambda b: (0, 0))],
        out_specs=[pl.BlockSpec((UBLK, 1), lambda b: (b, 0)),
                   pl.BlockSpec((UBLK, 1), lambda b: (b, 0))],
        out_shape=[jax.ShapeDtypeStruct((B, 1), jnp.float32),
                   jax.ShapeDtypeStruct((B, 1), jnp.float32)],
    )(pos_col, q1p, q2p)


def kernel(obs, actions,
           q1_W0, q1_b0, q1_W1, q1_b1, q1_W2, q1_b2, q1_W3, q1_b3,
           q2_W0, q2_b0, q2_W1, q2_b1, q2_W2, q2_b2, q2_W3, q2_b3):
    obs8 = obs[:, OBS_DIM - N_TASKS:].T
    pos2d, be2d, nv2d = _route(obs8)
    pos = pos2d.reshape(B)
    be = be2d.reshape(G)
    nv = nv2d.reshape(G)

    actions_pad = jnp.concatenate(
        [actions, jnp.zeros((B, ACT_PAD - ACT_DIM), jnp.float32)], axis=1)
    xso, xsa = _scatter(obs, actions_pad, pos.reshape(NW, NCH, CH))

    wts = []
    for (W0, b0, W1, b1, W2, b2, W3, b3) in (
            (q1_W0, q1_b0, q1_W1, q1_b1, q1_W2, q1_b2, q1_W3, q1_b3),
            (q2_W0, q2_b0, q2_W1, q2_b1, q2_W2, q2_b2, q2_W3, q2_b3)):
        wts += [W0[:, :OBS_DIM, :], W0[:, OBS_DIM:, :],
                b0.reshape(N_TASKS, 1, HID),
                W1, b1.reshape(N_TASKS, 1, HID),
                W2, b2.reshape(N_TASKS, 1, HID),
                W3, b3.reshape(N_TASKS, 1, 1)]
    q1p, q2p = _grouped_mlp(be, nv, xso, xsa, wts)

    return _unpermute(pos.reshape(B, 1), q1p, q2p)


# speedup vs baseline: 2.5558x; 1.2044x over previous
import functools

import jax
import jax.numpy as jnp
from jax import lax
from jax.experimental import pallas as pl
from jax.experimental.pallas import tpu as pltpu
from jax.experimental.pallas import tpu_sc as plsc

B = 4096
OBS_DIM = 1024
ACT_DIM = 32
N_TASKS = 8
HID = 1024
BLK = 256
G = B // BLK + N_TASKS
PAD = G * BLK

ACT_PAD = 128
NC = 2
NS = 16
NW = NC * NS
RPW = B // NW
CH = 32
NCH = RPW // CH


def _route_body(obs8_ref, pos_ref, be_ref, nv_ref):
    v = obs8_ref[...]
    best = v[0:1]
    idx = jnp.zeros((1, B), jnp.int32)
    for t in range(1, N_TASKS):
        m = v[t:t + 1] > best
        idx = jnp.where(m, t, idx)
        best = jnp.where(m, v[t:t + 1], best)
    tid = lax.broadcasted_iota(jnp.int32, (N_TASKS, B), 0)
    oh = (idx == tid).astype(jnp.float32)
    cs = oh
    s = 1
    while s < B:
        cs = cs + jnp.concatenate(
            [jnp.zeros((N_TASKS, s), jnp.float32), cs[:, :B - s]], axis=1)
        s *= 2
    counts = cs[:, B - 1:B]
    nb = jnp.floor((counts + (BLK - 1)) / BLK)
    inc = nb
    s = 1
    while s < N_TASKS:
        inc = inc + jnp.concatenate(
            [jnp.zeros((s, 1), jnp.float32), inc[:N_TASKS - s]], axis=0)
        s *= 2
    base = (inc - nb) * BLK
    posf = jnp.sum(oh * (base + cs - 1.0), axis=0, keepdims=True)
    pos_ref[...] = posf.astype(jnp.int32)
    bi = lax.broadcasted_iota(jnp.int32, (N_TASKS, G), 1)
    inc_i = inc.astype(jnp.int32)
    nb_i = nb.astype(jnp.int32)
    be = jnp.sum((bi >= inc_i).astype(jnp.int32), axis=0, keepdims=True)
    be_ref[...] = jnp.minimum(be, N_TASKS - 1)
    first = inc_i - nb_i
    owner = jnp.logical_and(bi >= first, bi < inc_i)
    vraw = counts.astype(jnp.int32) - (bi - first) * BLK
    vclamp = jnp.clip(vraw, 0, BLK)
    nv_ref[...] = jnp.sum(jnp.where(owner, vclamp, 0), axis=0, keepdims=True)


def _route(obs8):
    return pl.pallas_call(
        _route_body,
        out_shape=[jax.ShapeDtypeStruct((1, B), jnp.int32),
                   jax.ShapeDtypeStruct((1, G), jnp.int32),
                   jax.ShapeDtypeStruct((1, G), jnp.int32)],
    )(obs8)


def _scatter_body(obs_hbm, act_hbm, pos_hbm, xso_hbm, xsa_hbm,
                  idx_v, obuf, abuf, sem):
    wid = lax.axis_index("s") * NC + lax.axis_index("c")
    base = wid * RPW
    pltpu.sync_copy(pos_hbm.at[wid], idx_v)
    for ch in range(NCH):
        pltpu.sync_copy(obs_hbm.at[pl.ds(base + ch * CH, CH)], obuf)
        pltpu.async_copy(obuf, xso_hbm.at[idx_v.at[ch]], sem).wait()
        pltpu.sync_copy(act_hbm.at[pl.ds(base + ch * CH, CH)], abuf)
        pltpu.async_copy(abuf, xsa_hbm.at[idx_v.at[ch]], sem).wait()


def _scatter(obs, actions, pos3):
    return pl.kernel(
        _scatter_body,
        out_type=[jax.ShapeDtypeStruct((PAD, OBS_DIM), jnp.float32),
                  jax.ShapeDtypeStruct((PAD, ACT_PAD), jnp.float32)],
        mesh=plsc.VectorSubcoreMesh(core_axis_name="c", subcore_axis_name="s"),
        scratch_types=[
            pltpu.VMEM((NCH, CH), jnp.int32),
            pltpu.VMEM((CH, OBS_DIM), jnp.float32),
            pltpu.VMEM((CH, ACT_PAD), jnp.float32),
            pltpu.SemaphoreType.DMA,
        ],
    )(obs, actions, pos3)


def _mlp_body(be_ref, nv_ref, xo_ref, xa_ref,
              a_w0, a_b0, a_w1, a_b1, a_w2, a_b2, a_w3, a_b3,
              c_w0, c_b0, c_w1, c_b1, c_w2, c_b2, c_w3, c_b3,
              q1o_ref, q2o_ref):
    del be_ref
    valid = nv_ref[pl.program_id(0)]
    rmask = lax.broadcasted_iota(jnp.int32, (BLK, 1), 0) < valid
    xo = xo_ref[...]
    xa = xa_ref[...][:, :ACT_DIM]

    def head(w0, b0, w1, b1, w2, b2, w3, b3):
        h = (jnp.dot(xo, w0[0, :OBS_DIM], preferred_element_type=jnp.float32)
             + jnp.dot(xa, w0[0, OBS_DIM:], preferred_element_type=jnp.float32)
             + b0[0])
        h = jnp.maximum(h, 0.0)
        h = jnp.maximum(
            jnp.dot(h, w1[0], preferred_element_type=jnp.float32) + b1[0], 0.0)
        h = jnp.maximum(
            jnp.dot(h, w2[0], preferred_element_type=jnp.float32) + b2[0], 0.0)
        return jnp.dot(h, w3[0], preferred_element_type=jnp.float32) + b3[0]

    q1o_ref[...] = jnp.where(
        rmask, head(a_w0, a_b0, a_w1, a_b1, a_w2, a_b2, a_w3, a_b3), 0.0)
    q2o_ref[...] = jnp.where(
        rmask, head(c_w0, c_b0, c_w1, c_b1, c_w2, c_b2, c_w3, c_b3), 0.0)


def _grouped_mlp(be, nv, xso, xsa, wts):
    def xmap(b, eb, nv_):
        return (b, 0)

    def emap3(b, eb, nv_):
        return (eb[b], 0, 0)

    w_specs = []
    for _ in range(2):
        w_specs += [
            pl.BlockSpec((1, OBS_DIM + ACT_DIM, HID), emap3),
            pl.BlockSpec((1, 1, HID), emap3),
            pl.BlockSpec((1, HID, HID), emap3),
            pl.BlockSpec((1, 1, HID), emap3),
            pl.BlockSpec((1, HID, HID), emap3),
            pl.BlockSpec((1, 1, HID), emap3),
            pl.BlockSpec((1, HID, 1), emap3),
            pl.BlockSpec((1, 1, 1), emap3),
        ]
    grid_spec = pltpu.PrefetchScalarGridSpec(
        num_scalar_prefetch=2,
        grid=(G,),
        in_specs=[pl.BlockSpec((BLK, OBS_DIM), xmap),
                  pl.BlockSpec((BLK, ACT_PAD), xmap)] + w_specs,
        out_specs=[pl.BlockSpec((BLK, 1), xmap),
                   pl.BlockSpec((BLK, 1), xmap)],
    )
    return pl.pallas_call(
        _mlp_body,
        grid_spec=grid_spec,
        out_shape=[jax.ShapeDtypeStruct((PAD, 1), jnp.float32),
                   jax.ShapeDtypeStruct((PAD, 1), jnp.float32)],
    )(be, nv, xso, xsa, *wts)


UBLK = 256


def _unpermute_body(pos_ref, q1p_ref, q2p_ref, q1_ref, q2_ref):
    p = pos_ref[...]
    sel = (p == lax.broadcasted_iota(jnp.int32, (UBLK, PAD), 1)
           ).astype(jnp.float32)
    q1_ref[...] = jnp.dot(sel, q1p_ref[...],
                          preferred_element_type=jnp.float32)
    q2_ref[...] = jnp.dot(sel, q2p_ref[...],
                          preferred_element_type=jnp.float32)


def _unpermute(pos_col, q1p, q2p):
    return pl.pallas_call(
        _unpermute_body,
        grid=(B // UBLK,),
        in_specs=[pl.BlockSpec((UBLK, 1), lambda b: (b, 0)),
                  pl.BlockSpec((PAD, 1), lambda b: (0, 0)),
                  pl.BlockSpec((PAD, 1), lambda b: (0, 0))],
        out_specs=[pl.BlockSpec((UBLK, 1), lambda b: (b, 0)),
                   pl.BlockSpec((UBLK, 1), lambda b: (b, 0))],
        out_shape=[jax.ShapeDtypeStruct((B, 1), jnp.float32),
                   jax.ShapeDtypeStruct((B, 1), jnp.float32)],
    )(pos_col, q1p, q2p)


def kernel(obs, actions,
           q1_W0, q1_b0, q1_W1, q1_b1, q1_W2, q1_b2, q1_W3, q1_b3,
           q2_W0, q2_b0, q2_W1, q2_b1, q2_W2, q2_b2, q2_W3, q2_b3):
    obs8 = obs[:, OBS_DIM - N_TASKS:].T
    pos2d, be2d, nv2d = _route(obs8)
    pos = pos2d.reshape(B)
    be = be2d.reshape(G)
    nv = nv2d.reshape(G)

    actions_pad = jnp.concatenate(
        [actions, jnp.zeros((B, ACT_PAD - ACT_DIM), jnp.float32)], axis=1)
    xso, xsa = _scatter(obs, actions_pad, pos.reshape(NW, NCH, CH))

    wts = []
    for (W0, b0, W1, b1, W2, b2, W3, b3) in (
            (q1_W0, q1_b0, q1_W1, q1_b1, q1_W2, q1_b2, q1_W3, q1_b3),
            (q2_W0, q2_b0, q2_W1, q2_b1, q2_W2, q2_b2, q2_W3, q2_b3)):
        wts += [W0, b0.reshape(N_TASKS, 1, HID),
                W1, b1.reshape(N_TASKS, 1, HID),
                W2, b2.reshape(N_TASKS, 1, HID),
                W3, b3.reshape(N_TASKS, 1, 1)]
    q1p, q2p = _grouped_mlp(be, nv, xso, xsa, wts)

    return _unpermute(pos.reshape(B, 1), q1p, q2p)


# speedup vs baseline: 2.8211x; 1.1038x over previous
import functools

import jax
import jax.numpy as jnp
from jax import lax
from jax.experimental import pallas as pl
from jax.experimental.pallas import tpu as pltpu
from jax.experimental.pallas import tpu_sc as plsc

B = 4096
OBS_DIM = 1024
ACT_DIM = 32
N_TASKS = 8
HID = 1024
BLK = 256
G = B // BLK + N_TASKS
PAD = G * BLK

ACT_PAD = 128
QW = 128
NC = 2
NS = 16
NW = NC * NS
RPW = B // NW
CH = 32
NCH = RPW // CH


def _route_body(obs8_ref, pos_ref, be_ref, nv_ref):
    v = obs8_ref[...]
    best = v[0:1]
    idx = jnp.zeros((1, B), jnp.int32)
    for t in range(1, N_TASKS):
        m = v[t:t + 1] > best
        idx = jnp.where(m, t, idx)
        best = jnp.where(m, v[t:t + 1], best)
    tid = lax.broadcasted_iota(jnp.int32, (N_TASKS, B), 0)
    oh = (idx == tid).astype(jnp.float32)
    cs = oh
    s = 1
    while s < B:
        cs = cs + jnp.concatenate(
            [jnp.zeros((N_TASKS, s), jnp.float32), cs[:, :B - s]], axis=1)
        s *= 2
    counts = cs[:, B - 1:B]
    nb = jnp.floor((counts + (BLK - 1)) / BLK)
    inc = nb
    s = 1
    while s < N_TASKS:
        inc = inc + jnp.concatenate(
            [jnp.zeros((s, 1), jnp.float32), inc[:N_TASKS - s]], axis=0)
        s *= 2
    base = (inc - nb) * BLK
    posf = jnp.sum(oh * (base + cs - 1.0), axis=0, keepdims=True)
    pos_ref[...] = posf.astype(jnp.int32)
    bi = lax.broadcasted_iota(jnp.int32, (N_TASKS, G), 1)
    inc_i = inc.astype(jnp.int32)
    nb_i = nb.astype(jnp.int32)
    be = jnp.sum((bi >= inc_i).astype(jnp.int32), axis=0, keepdims=True)
    be_ref[...] = jnp.minimum(be, N_TASKS - 1)
    first = inc_i - nb_i
    owner = jnp.logical_and(bi >= first, bi < inc_i)
    vraw = counts.astype(jnp.int32) - (bi - first) * BLK
    vclamp = jnp.clip(vraw, 0, BLK)
    nv_ref[...] = jnp.sum(jnp.where(owner, vclamp, 0), axis=0, keepdims=True)


def _route(obs8):
    return pl.pallas_call(
        _route_body,
        out_shape=[jax.ShapeDtypeStruct((1, B), jnp.int32),
                   jax.ShapeDtypeStruct((1, G), jnp.int32),
                   jax.ShapeDtypeStruct((1, G), jnp.int32)],
    )(obs8)


def _scatter_body(obs_hbm, act_hbm, pos_hbm, xso_hbm, xsa_hbm,
                  idx_v, obuf, abuf, sem):
    wid = lax.axis_index("s") * NC + lax.axis_index("c")
    base = wid * RPW
    pltpu.sync_copy(pos_hbm.at[wid], idx_v)
    for ch in range(NCH):
        pltpu.sync_copy(obs_hbm.at[pl.ds(base + ch * CH, CH)], obuf)
        pltpu.async_copy(obuf, xso_hbm.at[idx_v.at[ch]], sem).wait()
        pltpu.sync_copy(act_hbm.at[pl.ds(base + ch * CH, CH)], abuf)
        pltpu.async_copy(abuf, xsa_hbm.at[idx_v.at[ch]], sem).wait()


def _scatter(obs, actions, pos3):
    return pl.kernel(
        _scatter_body,
        out_type=[jax.ShapeDtypeStruct((PAD, OBS_DIM), jnp.float32),
                  jax.ShapeDtypeStruct((PAD, ACT_PAD), jnp.float32)],
        mesh=plsc.VectorSubcoreMesh(core_axis_name="c", subcore_axis_name="s"),
        scratch_types=[
            pltpu.VMEM((NCH, CH), jnp.int32),
            pltpu.VMEM((CH, OBS_DIM), jnp.float32),
            pltpu.VMEM((CH, ACT_PAD), jnp.float32),
            pltpu.SemaphoreType.DMA,
        ],
    )(obs, actions, pos3)


def _mlp_body(be_ref, nv_ref, xo_ref, xa_ref,
              a_w0, a_b0, a_w1, a_b1, a_w2, a_b2, a_w3, a_b3,
              c_w0, c_b0, c_w1, c_b1, c_w2, c_b2, c_w3, c_b3,
              q1o_ref, q2o_ref):
    del be_ref
    valid = nv_ref[pl.program_id(0)]
    rmask = lax.broadcasted_iota(jnp.int32, (BLK, QW), 0) < valid
    xo = xo_ref[...]
    xa = xa_ref[...][:, :ACT_DIM]

    def head(w0, b0, w1, b1, w2, b2, w3, b3):
        h = (jnp.dot(xo, w0[0, :OBS_DIM], preferred_element_type=jnp.float32)
             + jnp.dot(xa, w0[0, OBS_DIM:], preferred_element_type=jnp.float32)
             + b0[0])
        h = jnp.maximum(h, 0.0)
        h = jnp.maximum(
            jnp.dot(h, w1[0], preferred_element_type=jnp.float32) + b1[0], 0.0)
        h = jnp.maximum(
            jnp.dot(h, w2[0], preferred_element_type=jnp.float32) + b2[0], 0.0)
        return jnp.dot(h, w3[0], preferred_element_type=jnp.float32) + b3[0]

    q1 = head(a_w0, a_b0, a_w1, a_b1, a_w2, a_b2, a_w3, a_b3)
    q2 = head(c_w0, c_b0, c_w1, c_b1, c_w2, c_b2, c_w3, c_b3)
    q1o_ref[...] = jnp.where(rmask, jnp.broadcast_to(q1, (BLK, QW)), 0.0)
    q2o_ref[...] = jnp.where(rmask, jnp.broadcast_to(q2, (BLK, QW)), 0.0)


def _grouped_mlp(be, nv, xso, xsa, wts):
    def xmap(b, eb, nv_):
        return (b, 0)

    def emap3(b, eb, nv_):
        return (eb[b], 0, 0)

    w_specs = []
    for _ in range(2):
        w_specs += [
            pl.BlockSpec((1, OBS_DIM + ACT_DIM, HID), emap3),
            pl.BlockSpec((1, 1, HID), emap3),
            pl.BlockSpec((1, HID, HID), emap3),
            pl.BlockSpec((1, 1, HID), emap3),
            pl.BlockSpec((1, HID, HID), emap3),
            pl.BlockSpec((1, 1, HID), emap3),
            pl.BlockSpec((1, HID, 1), emap3),
            pl.BlockSpec((1, 1, 1), emap3),
        ]
    grid_spec = pltpu.PrefetchScalarGridSpec(
        num_scalar_prefetch=2,
        grid=(G,),
        in_specs=[pl.BlockSpec((BLK, OBS_DIM), xmap),
                  pl.BlockSpec((BLK, ACT_PAD), xmap)] + w_specs,
        out_specs=[pl.BlockSpec((BLK, QW), xmap),
                   pl.BlockSpec((BLK, QW), xmap)],
    )
    return pl.pallas_call(
        _mlp_body,
        grid_spec=grid_spec,
        out_shape=[jax.ShapeDtypeStruct((PAD, QW), jnp.float32),
                   jax.ShapeDtypeStruct((PAD, QW), jnp.float32)],
    )(be, nv, xso, xsa, *wts)


def _gather_body(q1p_hbm, q2p_hbm, pos_hbm, q1_hbm, q2_hbm, idx_v, buf, sem):
    wid = lax.axis_index("s") * NC + lax.axis_index("c")
    base = wid * RPW
    pltpu.sync_copy(pos_hbm.at[wid], idx_v)
    pltpu.async_copy(q1p_hbm.at[idx_v], buf, sem).wait()
    pltpu.sync_copy(buf, q1_hbm.at[pl.ds(base, RPW)])
    pltpu.async_copy(q2p_hbm.at[idx_v], buf, sem).wait()
    pltpu.sync_copy(buf, q2_hbm.at[pl.ds(base, RPW)])


def _gather(q1p, q2p, pos128):
    return pl.kernel(
        _gather_body,
        out_type=[jax.ShapeDtypeStruct((B, QW), jnp.float32),
                  jax.ShapeDtypeStruct((B, QW), jnp.float32)],
        mesh=plsc.VectorSubcoreMesh(core_axis_name="c", subcore_axis_name="s"),
        scratch_types=[
            pltpu.VMEM((RPW,), jnp.int32),
            pltpu.VMEM((RPW, QW), jnp.float32),
            pltpu.SemaphoreType.DMA,
        ],
    )(q1p, q2p, pos128)


def kernel(obs, actions,
           q1_W0, q1_b0, q1_W1, q1_b1, q1_W2, q1_b2, q1_W3, q1_b3,
           q2_W0, q2_b0, q2_W1, q2_b1, q2_W2, q2_b2, q2_W3, q2_b3):
    obs8 = obs[:, OBS_DIM - N_TASKS:].T
    pos2d, be2d, nv2d = _route(obs8)
    pos = pos2d.reshape(B)
    be = be2d.reshape(G)
    nv = nv2d.reshape(G)

    actions_pad = jnp.concatenate(
        [actions, jnp.zeros((B, ACT_PAD - ACT_DIM), jnp.float32)], axis=1)
    xso, xsa = _scatter(obs, actions_pad, pos.reshape(NW, NCH, CH))

    wts = []
    for (W0, b0, W1, b1, W2, b2, W3, b3) in (
            (q1_W0, q1_b0, q1_W1, q1_b1, q1_W2, q1_b2, q1_W3, q1_b3),
            (q2_W0, q2_b0, q2_W1, q2_b1, q2_W2, q2_b2, q2_W3, q2_b3)):
        wts += [W0, b0.reshape(N_TASKS, 1, HID),
                W1, b1.reshape(N_TASKS, 1, HID),
                W2, b2.reshape(N_TASKS, 1, HID),
                W3, b3.reshape(N_TASKS, 1, 1)]
    q1p, q2p = _grouped_mlp(be, nv, xso, xsa, wts)

    q1w, q2w = _gather(q1p, q2p, pos.reshape(NW, RPW))
    return q1w[:, :1], q2w[:, :1]


# speedup vs baseline: 2.9849x; 1.0581x over previous
import functools

import jax
import jax.numpy as jnp
from jax import lax
from jax.experimental import pallas as pl
from jax.experimental.pallas import tpu as pltpu
from jax.experimental.pallas import tpu_sc as plsc

B = 4096
OBS_DIM = 1024
ACT_DIM = 32
N_TASKS = 8
HID = 1024
BLK = 256
G = B // BLK + N_TASKS
PAD = G * BLK

ACT_PAD = 128
QW = 128
NC = 2
NS = 16
NW = NC * NS
RPW = B // NW
CH = 32
NCH = RPW // CH


def _route_body(obs8_ref, pos_ref, be_ref, nv_ref):
    v = obs8_ref[...]
    best = v[0:1]
    idx = jnp.zeros((1, B), jnp.int32)
    for t in range(1, N_TASKS):
        m = v[t:t + 1] > best
        idx = jnp.where(m, t, idx)
        best = jnp.where(m, v[t:t + 1], best)
    tid = lax.broadcasted_iota(jnp.int32, (N_TASKS, B), 0)
    oh = (idx == tid).astype(jnp.float32)
    cs = oh
    s = 1
    while s < B:
        cs = cs + jnp.concatenate(
            [jnp.zeros((N_TASKS, s), jnp.float32), cs[:, :B - s]], axis=1)
        s *= 2
    counts = cs[:, B - 1:B]
    nb = jnp.floor((counts + (BLK - 1)) / BLK)
    inc = nb
    s = 1
    while s < N_TASKS:
        inc = inc + jnp.concatenate(
            [jnp.zeros((s, 1), jnp.float32), inc[:N_TASKS - s]], axis=0)
        s *= 2
    base = (inc - nb) * BLK
    posf = jnp.sum(oh * (base + cs - 1.0), axis=0, keepdims=True)
    pos_ref[...] = posf.astype(jnp.int32)
    bi = lax.broadcasted_iota(jnp.int32, (N_TASKS, G), 1)
    inc_i = inc.astype(jnp.int32)
    nb_i = nb.astype(jnp.int32)
    be = jnp.sum((bi >= inc_i).astype(jnp.int32), axis=0, keepdims=True)
    be_ref[...] = jnp.minimum(be, N_TASKS - 1)
    first = inc_i - nb_i
    owner = jnp.logical_and(bi >= first, bi < inc_i)
    vraw = counts.astype(jnp.int32) - (bi - first) * BLK
    vclamp = jnp.clip(vraw, 0, BLK)
    nv_ref[...] = jnp.sum(jnp.where(owner, vclamp, 0), axis=0, keepdims=True)


def _route(obs8):
    return pl.pallas_call(
        _route_body,
        out_shape=[jax.ShapeDtypeStruct((1, B), jnp.int32),
                   jax.ShapeDtypeStruct((1, G), jnp.int32),
                   jax.ShapeDtypeStruct((1, G), jnp.int32)],
    )(obs8)


def _scatter_body(obs_hbm, act_hbm, pos_hbm, xso_hbm, xsa_hbm,
                  idx_v, obuf, abuf, sem):
    wid = lax.axis_index("s") * NC + lax.axis_index("c")
    base = wid * RPW
    pltpu.sync_copy(pos_hbm.at[wid], idx_v)
    for ch in range(NCH):
        pltpu.sync_copy(obs_hbm.at[pl.ds(base + ch * CH, CH)], obuf)
        pltpu.async_copy(obuf, xso_hbm.at[idx_v.at[ch]], sem).wait()
        pltpu.sync_copy(act_hbm.at[pl.ds(base + ch * CH, CH)], abuf)
        pltpu.async_copy(abuf, xsa_hbm.at[idx_v.at[ch]], sem).wait()


def _scatter(obs, actions, pos3):
    return pl.kernel(
        _scatter_body,
        out_type=[jax.ShapeDtypeStruct((PAD, OBS_DIM), jnp.float32),
                  jax.ShapeDtypeStruct((PAD, ACT_PAD), jnp.float32)],
        mesh=plsc.VectorSubcoreMesh(core_axis_name="c", subcore_axis_name="s"),
        scratch_types=[
            pltpu.VMEM((NCH, CH), jnp.int32),
            pltpu.VMEM((CH, OBS_DIM), jnp.float32),
            pltpu.VMEM((CH, ACT_PAD), jnp.float32),
            pltpu.SemaphoreType.DMA,
        ],
    )(obs, actions, pos3)


def _mlp_body(be_ref, nv_ref, xo_ref, xa_ref,
              a_w0, a_b0, a_w1, a_b1, a_w2, a_b2, a_w3, a_b3,
              c_w0, c_b0, c_w1, c_b1, c_w2, c_b2, c_w3, c_b3,
              q1o_ref, q2o_ref):
    del be_ref
    valid = nv_ref[pl.program_id(0)]

    @pl.when(valid > 0)
    def _():
        rmask = lax.broadcasted_iota(jnp.int32, (BLK, QW), 0) < valid
        xo = xo_ref[...]
        xa = xa_ref[...][:, :ACT_DIM]

        def head(w0, b0, w1, b1, w2, b2, w3, b3):
            h = (jnp.dot(xo, w0[0, :OBS_DIM],
                         preferred_element_type=jnp.float32)
                 + jnp.dot(xa, w0[0, OBS_DIM:],
                           preferred_element_type=jnp.float32)
                 + b0[0])
            h = jnp.maximum(h, 0.0)
            h = jnp.maximum(
                jnp.dot(h, w1[0], preferred_element_type=jnp.float32)
                + b1[0], 0.0)
            h = jnp.maximum(
                jnp.dot(h, w2[0], preferred_element_type=jnp.float32)
                + b2[0], 0.0)
            return jnp.dot(h, w3[0], preferred_element_type=jnp.float32) + b3[0]

        q1 = head(a_w0, a_b0, a_w1, a_b1, a_w2, a_b2, a_w3, a_b3)
        q2 = head(c_w0, c_b0, c_w1, c_b1, c_w2, c_b2, c_w3, c_b3)
        q1o_ref[...] = jnp.where(rmask, jnp.broadcast_to(q1, (BLK, QW)), 0.0)
        q2o_ref[...] = jnp.where(rmask, jnp.broadcast_to(q2, (BLK, QW)), 0.0)


def _grouped_mlp(be, nv, xso, xsa, wts):
    def xmap(b, eb, nv_):
        return (b, 0)

    def emap3(b, eb, nv_):
        return (eb[b], 0, 0)

    w_specs = []
    for _ in range(2):
        w_specs += [
            pl.BlockSpec((1, OBS_DIM + ACT_DIM, HID), emap3),
            pl.BlockSpec((1, 1, HID), emap3),
            pl.BlockSpec((1, HID, HID), emap3),
            pl.BlockSpec((1, 1, HID), emap3),
            pl.BlockSpec((1, HID, HID), emap3),
            pl.BlockSpec((1, 1, HID), emap3),
            pl.BlockSpec((1, HID, 1), emap3),
            pl.BlockSpec((1, 1, 1), emap3),
        ]
    grid_spec = pltpu.PrefetchScalarGridSpec(
        num_scalar_prefetch=2,
        grid=(G,),
        in_specs=[pl.BlockSpec((BLK, OBS_DIM), xmap),
                  pl.BlockSpec((BLK, ACT_PAD), xmap)] + w_specs,
        out_specs=[pl.BlockSpec((BLK, QW), xmap),
                   pl.BlockSpec((BLK, QW), xmap)],
    )
    return pl.pallas_call(
        _mlp_body,
        grid_spec=grid_spec,
        out_shape=[jax.ShapeDtypeStruct((PAD, QW), jnp.float32),
                   jax.ShapeDtypeStruct((PAD, QW), jnp.float32)],
    )(be, nv, xso, xsa, *wts)


def _gather_body(q1p_hbm, q2p_hbm, pos_hbm, q1_hbm, q2_hbm, idx_v, buf, sem):
    wid = lax.axis_index("s") * NC + lax.axis_index("c")
    base = wid * RPW
    pltpu.sync_copy(pos_hbm.at[wid], idx_v)
    pltpu.async_copy(q1p_hbm.at[idx_v], buf, sem).wait()
    pltpu.sync_copy(buf, q1_hbm.at[pl.ds(base, RPW)])
    pltpu.async_copy(q2p_hbm.at[idx_v], buf, sem).wait()
    pltpu.sync_copy(buf, q2_hbm.at[pl.ds(base, RPW)])


def _gather(q1p, q2p, pos128):
    return pl.kernel(
        _gather_body,
        out_type=[jax.ShapeDtypeStruct((B, QW), jnp.float32),
                  jax.ShapeDtypeStruct((B, QW), jnp.float32)],
        mesh=plsc.VectorSubcoreMesh(core_axis_name="c", subcore_axis_name="s"),
        scratch_types=[
            pltpu.VMEM((RPW,), jnp.int32),
            pltpu.VMEM((RPW, QW), jnp.float32),
            pltpu.SemaphoreType.DMA,
        ],
    )(q1p, q2p, pos128)


def kernel(obs, actions,
           q1_W0, q1_b0, q1_W1, q1_b1, q1_W2, q1_b2, q1_W3, q1_b3,
           q2_W0, q2_b0, q2_W1, q2_b1, q2_W2, q2_b2, q2_W3, q2_b3):
    obs8 = obs[:, OBS_DIM - N_TASKS:].T
    pos2d, be2d, nv2d = _route(obs8)
    pos = pos2d.reshape(B)
    be = be2d.reshape(G)
    nv = nv2d.reshape(G)

    actions_pad = jnp.concatenate(
        [actions, jnp.zeros((B, ACT_PAD - ACT_DIM), jnp.float32)], axis=1)
    xso, xsa = _scatter(obs, actions_pad, pos.reshape(NW, NCH, CH))

    wts = []
    for (W0, b0, W1, b1, W2, b2, W3, b3) in (
            (q1_W0, q1_b0, q1_W1, q1_b1, q1_W2, q1_b2, q1_W3, q1_b3),
            (q2_W0, q2_b0, q2_W1, q2_b1, q2_W2, q2_b2, q2_W3, q2_b3)):
        wts += [W0, b0.reshape(N_TASKS, 1, HID),
                W1, b1.reshape(N_TASKS, 1, HID),
                W2, b2.reshape(N_TASKS, 1, HID),
                W3, b3.reshape(N_TASKS, 1, 1)]
    q1p, q2p = _grouped_mlp(be, nv, xso, xsa, wts)

    q1w, q2w = _gather(q1p, q2p, pos.reshape(NW, RPW))
    return q1w[:, :1], q2w[:, :1]


# speedup vs baseline: 3.1924x; 1.0695x over previous
import functools

import jax
import jax.numpy as jnp
from jax import lax
from jax.experimental import pallas as pl
from jax.experimental.pallas import tpu as pltpu
from jax.experimental.pallas import tpu_sc as plsc

B = 4096
OBS_DIM = 1024
ACT_DIM = 32
N_TASKS = 8
HID = 1024
BLK = 256
G = B // BLK + N_TASKS
PAD = G * BLK

ACT_PAD = 128
QW = 128
NC = 2
NS = 16
NW = NC * NS
RPW = B // NW
CH = 32
NCH = RPW // CH


def _route_body(obs8_ref, pos_ref, be_ref, nv_ref):
    v = obs8_ref[...]
    best = v[0:1]
    idx = jnp.zeros((1, B), jnp.int32)
    for t in range(1, N_TASKS):
        m = v[t:t + 1] > best
        idx = jnp.where(m, t, idx)
        best = jnp.where(m, v[t:t + 1], best)
    tid = lax.broadcasted_iota(jnp.int32, (N_TASKS, B), 0)
    oh = (idx == tid).astype(jnp.float32)
    cs = oh
    s = 1
    while s < B:
        cs = cs + jnp.concatenate(
            [jnp.zeros((N_TASKS, s), jnp.float32), cs[:, :B - s]], axis=1)
        s *= 2
    counts = cs[:, B - 1:B]
    nb = jnp.floor((counts + (BLK - 1)) / BLK)
    inc = nb
    s = 1
    while s < N_TASKS:
        inc = inc + jnp.concatenate(
            [jnp.zeros((s, 1), jnp.float32), inc[:N_TASKS - s]], axis=0)
        s *= 2
    base = (inc - nb) * BLK
    posf = jnp.sum(oh * (base + cs - 1.0), axis=0, keepdims=True)
    pos_ref[...] = posf.astype(jnp.int32)
    bi = lax.broadcasted_iota(jnp.int32, (N_TASKS, G), 1)
    inc_i = inc.astype(jnp.int32)
    nb_i = nb.astype(jnp.int32)
    be = jnp.sum((bi >= inc_i).astype(jnp.int32), axis=0, keepdims=True)
    be_ref[...] = jnp.minimum(be, N_TASKS - 1)
    first = inc_i - nb_i
    owner = jnp.logical_and(bi >= first, bi < inc_i)
    vraw = counts.astype(jnp.int32) - (bi - first) * BLK
    vclamp = jnp.clip(vraw, 0, BLK)
    nv_ref[...] = jnp.sum(jnp.where(owner, vclamp, 0), axis=0, keepdims=True)


def _route(obs8):
    return pl.pallas_call(
        _route_body,
        out_shape=[jax.ShapeDtypeStruct((1, B), jnp.int32),
                   jax.ShapeDtypeStruct((1, G), jnp.int32),
                   jax.ShapeDtypeStruct((1, G), jnp.int32)],
    )(obs8)


def _scatter_body(obs_hbm, act_hbm, pos_hbm, xso_hbm, xsa_hbm,
                  idx_v, obuf, abuf, sem):
    wid = lax.axis_index("s") * NC + lax.axis_index("c")
    base = wid * RPW
    pltpu.sync_copy(pos_hbm.at[wid], idx_v)
    for ch in range(NCH):
        pltpu.sync_copy(obs_hbm.at[pl.ds(base + ch * CH, CH)], obuf)
        pltpu.async_copy(obuf, xso_hbm.at[idx_v.at[ch]], sem).wait()
        pltpu.sync_copy(act_hbm.at[pl.ds(base + ch * CH, CH)], abuf)
        pltpu.async_copy(abuf, xsa_hbm.at[idx_v.at[ch]], sem).wait()


def _scatter(obs, actions, pos3):
    return pl.kernel(
        _scatter_body,
        out_type=[jax.ShapeDtypeStruct((PAD, OBS_DIM), jnp.float32),
                  jax.ShapeDtypeStruct((PAD, ACT_PAD), jnp.float32)],
        mesh=plsc.VectorSubcoreMesh(core_axis_name="c", subcore_axis_name="s"),
        scratch_types=[
            pltpu.VMEM((NCH, CH), jnp.int32),
            pltpu.VMEM((CH, OBS_DIM), jnp.float32),
            pltpu.VMEM((CH, ACT_PAD), jnp.float32),
            pltpu.SemaphoreType.DMA,
        ],
    )(obs, actions, pos3)


def _mlp_body(be_ref, nv_ref, xo_ref, xa_ref,
              a_w0, a_b0, a_w1, a_b1, a_w2, a_b2, a_w3, a_b3,
              c_w0, c_b0, c_w1, c_b1, c_w2, c_b2, c_w3, c_b3,
              q1o_ref, q2o_ref):
    del be_ref
    valid = nv_ref[pl.program_id(0)]

    @pl.when(valid > 0)
    def _():
        rmask = lax.broadcasted_iota(jnp.int32, (BLK, QW), 0) < valid
        x = jnp.concatenate([xo_ref[...], xa_ref[...][:, :ACT_DIM]], axis=1)

        def head(w0, b0, w1, b1, w2, b2, w3, b3):
            h = jnp.dot(x, w0[0], preferred_element_type=jnp.float32) + b0[0]
            h = jnp.maximum(h, 0.0)
            h = jnp.maximum(
                jnp.dot(h, w1[0], preferred_element_type=jnp.float32)
                + b1[0], 0.0)
            h = jnp.maximum(
                jnp.dot(h, w2[0], preferred_element_type=jnp.float32)
                + b2[0], 0.0)
            return jnp.dot(h, w3[0], preferred_element_type=jnp.float32) + b3[0]

        q1 = head(a_w0, a_b0, a_w1, a_b1, a_w2, a_b2, a_w3, a_b3)
        q2 = head(c_w0, c_b0, c_w1, c_b1, c_w2, c_b2, c_w3, c_b3)
        q1o_ref[...] = jnp.where(rmask, jnp.broadcast_to(q1, (BLK, QW)), 0.0)
        q2o_ref[...] = jnp.where(rmask, jnp.broadcast_to(q2, (BLK, QW)), 0.0)


def _grouped_mlp(be, nv, xso, xsa, wts):
    def xmap(b, eb, nv_):
        return (b, 0)

    def emap3(b, eb, nv_):
        return (eb[b], 0, 0)

    w_specs = []
    for _ in range(2):
        w_specs += [
            pl.BlockSpec((1, OBS_DIM + ACT_DIM, HID), emap3),
            pl.BlockSpec((1, 1, HID), emap3),
            pl.BlockSpec((1, HID, HID), emap3),
            pl.BlockSpec((1, 1, HID), emap3),
            pl.BlockSpec((1, HID, HID), emap3),
            pl.BlockSpec((1, 1, HID), emap3),
            pl.BlockSpec((1, HID, 1), emap3),
            pl.BlockSpec((1, 1, 1), emap3),
        ]
    grid_spec = pltpu.PrefetchScalarGridSpec(
        num_scalar_prefetch=2,
        grid=(G,),
        in_specs=[pl.BlockSpec((BLK, OBS_DIM), xmap),
                  pl.BlockSpec((BLK, ACT_PAD), xmap)] + w_specs,
        out_specs=[pl.BlockSpec((BLK, QW), xmap),
                   pl.BlockSpec((BLK, QW), xmap)],
    )
    return pl.pallas_call(
        _mlp_body,
        grid_spec=grid_spec,
        out_shape=[jax.ShapeDtypeStruct((PAD, QW), jnp.float32),
                   jax.ShapeDtypeStruct((PAD, QW), jnp.float32)],
    )(be, nv, xso, xsa, *wts)


def _gather_body(q1p_hbm, q2p_hbm, pos_hbm, q1_hbm, q2_hbm, idx_v, buf, sem):
    wid = lax.axis_index("s") * NC + lax.axis_index("c")
    base = wid * RPW
    pltpu.sync_copy(pos_hbm.at[wid], idx_v)
    pltpu.async_copy(q1p_hbm.at[idx_v], buf, sem).wait()
    pltpu.sync_copy(buf, q1_hbm.at[pl.ds(base, RPW)])
    pltpu.async_copy(q2p_hbm.at[idx_v], buf, sem).wait()
    pltpu.sync_copy(buf, q2_hbm.at[pl.ds(base, RPW)])


def _gather(q1p, q2p, pos128):
    return pl.kernel(
        _gather_body,
        out_type=[jax.ShapeDtypeStruct((B, QW), jnp.float32),
                  jax.ShapeDtypeStruct((B, QW), jnp.float32)],
        mesh=plsc.VectorSubcoreMesh(core_axis_name="c", subcore_axis_name="s"),
        scratch_types=[
            pltpu.VMEM((RPW,), jnp.int32),
            pltpu.VMEM((RPW, QW), jnp.float32),
            pltpu.SemaphoreType.DMA,
        ],
    )(q1p, q2p, pos128)


def kernel(obs, actions,
           q1_W0, q1_b0, q1_W1, q1_b1, q1_W2, q1_b2, q1_W3, q1_b3,
           q2_W0, q2_b0, q2_W1, q2_b1, q2_W2, q2_b2, q2_W3, q2_b3):
    obs8 = obs[:, OBS_DIM - N_TASKS:].T
    pos2d, be2d, nv2d = _route(obs8)
    pos = pos2d.reshape(B)
    be = be2d.reshape(G)
    nv = nv2d.reshape(G)

    actions_pad = jnp.concatenate(
        [actions, jnp.zeros((B, ACT_PAD - ACT_DIM), jnp.float32)], axis=1)
    xso, xsa = _scatter(obs, actions_pad, pos.reshape(NW, NCH, CH))

    wts = []
    for (W0, b0, W1, b1, W2, b2, W3, b3) in (
            (q1_W0, q1_b0, q1_W1, q1_b1, q1_W2, q1_b2, q1_W3, q1_b3),
            (q2_W0, q2_b0, q2_W1, q2_b1, q2_W2, q2_b2, q2_W3, q2_b3)):
        wts += [W0, b0.reshape(N_TASKS, 1, HID),
                W1, b1.reshape(N_TASKS, 1, HID),
                W2, b2.reshape(N_TASKS, 1, HID),
                W3, b3.reshape(N_TASKS, 1, 1)]
    q1p, q2p = _grouped_mlp(be, nv, xso, xsa, wts)

    q1w, q2w = _gather(q1p, q2p, pos.reshape(NW, RPW))
    return q1w[:, :1], q2w[:, :1]


# speedup vs baseline: 3.2326x; 1.0126x over previous
import functools

import jax
import jax.numpy as jnp
from jax import lax
from jax.experimental import pallas as pl
from jax.experimental.pallas import tpu as pltpu
from jax.experimental.pallas import tpu_sc as plsc

B = 4096
OBS_DIM = 1024
ACT_DIM = 32
N_TASKS = 8
HID = 1024
BLK = 256
G = B // BLK + N_TASKS
PAD = G * BLK

ACT_PAD = 128
QW = 128
NC = 2
NS = 16
NW = NC * NS
RPW = B // NW
CH = 32
NCH = RPW // CH


def _route_body(obs8_ref, pos_ref, be_ref, nv_ref):
    v = obs8_ref[...]
    best = v[0:1]
    idx = jnp.zeros((1, B), jnp.int32)
    for t in range(1, N_TASKS):
        m = v[t:t + 1] > best
        idx = jnp.where(m, t, idx)
        best = jnp.where(m, v[t:t + 1], best)
    tid = lax.broadcasted_iota(jnp.int32, (N_TASKS, B), 0)
    oh = (idx == tid).astype(jnp.float32)
    cs = oh
    s = 1
    while s < B:
        cs = cs + jnp.concatenate(
            [jnp.zeros((N_TASKS, s), jnp.float32), cs[:, :B - s]], axis=1)
        s *= 2
    counts = cs[:, B - 1:B]
    nb = jnp.floor((counts + (BLK - 1)) / BLK)
    inc = nb
    s = 1
    while s < N_TASKS:
        inc = inc + jnp.concatenate(
            [jnp.zeros((s, 1), jnp.float32), inc[:N_TASKS - s]], axis=0)
        s *= 2
    base = (inc - nb) * BLK
    posf = jnp.sum(oh * (base + cs - 1.0), axis=0, keepdims=True)
    pos_ref[...] = posf.astype(jnp.int32)
    bi = lax.broadcasted_iota(jnp.int32, (N_TASKS, G), 1)
    inc_i = inc.astype(jnp.int32)
    nb_i = nb.astype(jnp.int32)
    be = jnp.sum((bi >= inc_i).astype(jnp.int32), axis=0, keepdims=True)
    be_ref[...] = jnp.minimum(be, N_TASKS - 1)
    first = inc_i - nb_i
    owner = jnp.logical_and(bi >= first, bi < inc_i)
    vraw = counts.astype(jnp.int32) - (bi - first) * BLK
    vclamp = jnp.clip(vraw, 0, BLK)
    nv_ref[...] = jnp.sum(jnp.where(owner, vclamp, 0), axis=0, keepdims=True)


def _route(obs8):
    return pl.pallas_call(
        _route_body,
        out_shape=[jax.ShapeDtypeStruct((1, B), jnp.int32),
                   jax.ShapeDtypeStruct((1, G), jnp.int32),
                   jax.ShapeDtypeStruct((1, G), jnp.int32)],
    )(obs8)


def _scatter_body(obs_hbm, act_hbm, pos_hbm, xso_hbm, xsa_hbm,
                  idx_v, obuf, abuf, sem):
    wid = lax.axis_index("s") * NC + lax.axis_index("c")
    base = wid * RPW
    pltpu.sync_copy(pos_hbm.at[wid], idx_v)
    for ch in range(NCH):
        pltpu.sync_copy(obs_hbm.at[pl.ds(base + ch * CH, CH)], obuf)
        pltpu.async_copy(obuf, xso_hbm.at[idx_v.at[ch]], sem).wait()
        pltpu.sync_copy(act_hbm.at[pl.ds(base + ch * CH, CH)], abuf)
        pltpu.async_copy(abuf, xsa_hbm.at[idx_v.at[ch]], sem).wait()


def _scatter(obs, actions, pos3):
    return pl.kernel(
        _scatter_body,
        out_type=[jax.ShapeDtypeStruct((PAD, OBS_DIM), jnp.float32),
                  jax.ShapeDtypeStruct((PAD, ACT_PAD), jnp.float32)],
        mesh=plsc.VectorSubcoreMesh(core_axis_name="c", subcore_axis_name="s"),
        scratch_types=[
            pltpu.VMEM((NCH, CH), jnp.int32),
            pltpu.VMEM((CH, OBS_DIM), jnp.float32),
            pltpu.VMEM((CH, ACT_PAD), jnp.float32),
            pltpu.SemaphoreType.DMA,
        ],
    )(obs, actions, pos3)


def _mlp_body(be_ref, nv_ref, xo_ref, xa_ref,
              a_w0, a_b0, a_w1, a_b1, a_w2, a_b2, a_w3, a_b3,
              c_w0, c_b0, c_w1, c_b1, c_w2, c_b2, c_w3, c_b3,
              q1o_ref, q2o_ref):
    del be_ref
    valid = nv_ref[pl.program_id(0)]

    @pl.when(valid > 0)
    def _():
        rmask = lax.broadcasted_iota(jnp.int32, (BLK, QW), 0) < valid
        x = jnp.concatenate([xo_ref[...], xa_ref[...][:, :ACT_DIM]], axis=1)

        bf = jnp.bfloat16

        def head(w0, b0, w1, b1, w2, b2, w3, b3):
            h = jnp.dot(x.astype(bf), w0[0].astype(bf),
                        preferred_element_type=jnp.float32) + b0[0]
            h = jnp.maximum(h, 0.0)
            h = jnp.maximum(
                jnp.dot(h.astype(bf), w1[0].astype(bf),
                        preferred_element_type=jnp.float32) + b1[0], 0.0)
            h = jnp.maximum(
                jnp.dot(h.astype(bf), w2[0].astype(bf),
                        preferred_element_type=jnp.float32) + b2[0], 0.0)
            return jnp.dot(h.astype(bf), w3[0].astype(bf),
                           preferred_element_type=jnp.float32) + b3[0]

        q1 = head(a_w0, a_b0, a_w1, a_b1, a_w2, a_b2, a_w3, a_b3)
        q2 = head(c_w0, c_b0, c_w1, c_b1, c_w2, c_b2, c_w3, c_b3)
        q1o_ref[...] = jnp.where(rmask, jnp.broadcast_to(q1, (BLK, QW)), 0.0)
        q2o_ref[...] = jnp.where(rmask, jnp.broadcast_to(q2, (BLK, QW)), 0.0)


def _grouped_mlp(be, nv, xso, xsa, wts):
    def xmap(b, eb, nv_):
        return (b, 0)

    def emap3(b, eb, nv_):
        return (eb[b], 0, 0)

    w_specs = []
    for _ in range(2):
        w_specs += [
            pl.BlockSpec((1, OBS_DIM + ACT_DIM, HID), emap3),
            pl.BlockSpec((1, 1, HID), emap3),
            pl.BlockSpec((1, HID, HID), emap3),
            pl.BlockSpec((1, 1, HID), emap3),
            pl.BlockSpec((1, HID, HID), emap3),
            pl.BlockSpec((1, 1, HID), emap3),
            pl.BlockSpec((1, HID, 1), emap3),
            pl.BlockSpec((1, 1, 1), emap3),
        ]
    grid_spec = pltpu.PrefetchScalarGridSpec(
        num_scalar_prefetch=2,
        grid=(G,),
        in_specs=[pl.BlockSpec((BLK, OBS_DIM), xmap),
                  pl.BlockSpec((BLK, ACT_PAD), xmap)] + w_specs,
        out_specs=[pl.BlockSpec((BLK, QW), xmap),
                   pl.BlockSpec((BLK, QW), xmap)],
    )
    return pl.pallas_call(
        _mlp_body,
        grid_spec=grid_spec,
        out_shape=[jax.ShapeDtypeStruct((PAD, QW), jnp.float32),
                   jax.ShapeDtypeStruct((PAD, QW), jnp.float32)],
    )(be, nv, xso, xsa, *wts)


def _gather_body(q1p_hbm, q2p_hbm, pos_hbm, q1_hbm, q2_hbm, idx_v, buf, sem):
    wid = lax.axis_index("s") * NC + lax.axis_index("c")
    base = wid * RPW
    pltpu.sync_copy(pos_hbm.at[wid], idx_v)
    pltpu.async_copy(q1p_hbm.at[idx_v], buf, sem).wait()
    pltpu.sync_copy(buf, q1_hbm.at[pl.ds(base, RPW)])
    pltpu.async_copy(q2p_hbm.at[idx_v], buf, sem).wait()
    pltpu.sync_copy(buf, q2_hbm.at[pl.ds(base, RPW)])


def _gather(q1p, q2p, pos128):
    return pl.kernel(
        _gather_body,
        out_type=[jax.ShapeDtypeStruct((B, QW), jnp.float32),
                  jax.ShapeDtypeStruct((B, QW), jnp.float32)],
        mesh=plsc.VectorSubcoreMesh(core_axis_name="c", subcore_axis_name="s"),
        scratch_types=[
            pltpu.VMEM((RPW,), jnp.int32),
            pltpu.VMEM((RPW, QW), jnp.float32),
            pltpu.SemaphoreType.DMA,
        ],
    )(q1p, q2p, pos128)


def kernel(obs, actions,
           q1_W0, q1_b0, q1_W1, q1_b1, q1_W2, q1_b2, q1_W3, q1_b3,
           q2_W0, q2_b0, q2_W1, q2_b1, q2_W2, q2_b2, q2_W3, q2_b3):
    obs8 = obs[:, OBS_DIM - N_TASKS:].T
    pos2d, be2d, nv2d = _route(obs8)
    pos = pos2d.reshape(B)
    be = be2d.reshape(G)
    nv = nv2d.reshape(G)

    actions_pad = jnp.concatenate(
        [actions, jnp.zeros((B, ACT_PAD - ACT_DIM), jnp.float32)], axis=1)
    xso, xsa = _scatter(obs, actions_pad, pos.reshape(NW, NCH, CH))

    wts = []
    for (W0, b0, W1, b1, W2, b2, W3, b3) in (
            (q1_W0, q1_b0, q1_W1, q1_b1, q1_W2, q1_b2, q1_W3, q1_b3),
            (q2_W0, q2_b0, q2_W1, q2_b1, q2_W2, q2_b2, q2_W3, q2_b3)):
        wts += [W0, b0.reshape(N_TASKS, 1, HID),
                W1, b1.reshape(N_TASKS, 1, HID),
                W2, b2.reshape(N_TASKS, 1, HID),
                W3, b3.reshape(N_TASKS, 1, 1)]
    q1p, q2p = _grouped_mlp(be, nv, xso, xsa, wts)

    q1w, q2w = _gather(q1p, q2p, pos.reshape(NW, RPW))
    return q1w[:, :1], q2w[:, :1]


# speedup vs baseline: 3.4549x; 1.0688x over previous
import functools

import jax
import jax.numpy as jnp
from jax import lax
from jax.experimental import pallas as pl
from jax.experimental.pallas import tpu as pltpu
from jax.experimental.pallas import tpu_sc as plsc

B = 4096
OBS_DIM = 1024
ACT_DIM = 32
N_TASKS = 8
HID = 1024
BLK = 512
G = B // BLK + N_TASKS
PAD = G * BLK

ACT_PAD = 128
QW = 128
NC = 2
NS = 16
NW = NC * NS
RPW = B // NW
CH = 32
NCH = RPW // CH


def _route_body(obs8_ref, pos_ref, be_ref, nv_ref):
    v = obs8_ref[...]
    best = v[0:1]
    idx = jnp.zeros((1, B), jnp.int32)
    for t in range(1, N_TASKS):
        m = v[t:t + 1] > best
        idx = jnp.where(m, t, idx)
        best = jnp.where(m, v[t:t + 1], best)
    tid = lax.broadcasted_iota(jnp.int32, (N_TASKS, B), 0)
    oh = (idx == tid).astype(jnp.float32)
    cs = oh
    s = 1
    while s < B:
        cs = cs + jnp.concatenate(
            [jnp.zeros((N_TASKS, s), jnp.float32), cs[:, :B - s]], axis=1)
        s *= 2
    counts = cs[:, B - 1:B]
    nb = jnp.floor((counts + (BLK - 1)) / BLK)
    inc = nb
    s = 1
    while s < N_TASKS:
        inc = inc + jnp.concatenate(
            [jnp.zeros((s, 1), jnp.float32), inc[:N_TASKS - s]], axis=0)
        s *= 2
    base = (inc - nb) * BLK
    posf = jnp.sum(oh * (base + cs - 1.0), axis=0, keepdims=True)
    pos_ref[...] = posf.astype(jnp.int32)
    bi = lax.broadcasted_iota(jnp.int32, (N_TASKS, G), 1)
    inc_i = inc.astype(jnp.int32)
    nb_i = nb.astype(jnp.int32)
    be = jnp.sum((bi >= inc_i).astype(jnp.int32), axis=0, keepdims=True)
    be_ref[...] = jnp.minimum(be, N_TASKS - 1)
    first = inc_i - nb_i
    owner = jnp.logical_and(bi >= first, bi < inc_i)
    vraw = counts.astype(jnp.int32) - (bi - first) * BLK
    vclamp = jnp.clip(vraw, 0, BLK)
    nv_ref[...] = jnp.sum(jnp.where(owner, vclamp, 0), axis=0, keepdims=True)


def _route(obs8):
    return pl.pallas_call(
        _route_body,
        out_shape=[jax.ShapeDtypeStruct((1, B), jnp.int32),
                   jax.ShapeDtypeStruct((1, G), jnp.int32),
                   jax.ShapeDtypeStruct((1, G), jnp.int32)],
    )(obs8)


def _scatter_body(obs_hbm, act_hbm, pos_hbm, xso_hbm, xsa_hbm,
                  idx_v, obuf, abuf, sem):
    wid = lax.axis_index("s") * NC + lax.axis_index("c")
    base = wid * RPW
    pltpu.sync_copy(pos_hbm.at[wid], idx_v)
    for ch in range(NCH):
        pltpu.sync_copy(obs_hbm.at[pl.ds(base + ch * CH, CH)], obuf)
        pltpu.async_copy(obuf, xso_hbm.at[idx_v.at[ch]], sem).wait()
        pltpu.sync_copy(act_hbm.at[pl.ds(base + ch * CH, CH)], abuf)
        pltpu.async_copy(abuf, xsa_hbm.at[idx_v.at[ch]], sem).wait()


def _scatter(obs, actions, pos3):
    return pl.kernel(
        _scatter_body,
        out_type=[jax.ShapeDtypeStruct((PAD, OBS_DIM), jnp.float32),
                  jax.ShapeDtypeStruct((PAD, ACT_PAD), jnp.float32)],
        mesh=plsc.VectorSubcoreMesh(core_axis_name="c", subcore_axis_name="s"),
        scratch_types=[
            pltpu.VMEM((NCH, CH), jnp.int32),
            pltpu.VMEM((CH, OBS_DIM), jnp.float32),
            pltpu.VMEM((CH, ACT_PAD), jnp.float32),
            pltpu.SemaphoreType.DMA,
        ],
    )(obs, actions, pos3)


def _mlp_body(be_ref, nv_ref, xo_ref, xa_ref,
              a_w0, a_b0, a_w1, a_b1, a_w2, a_b2, a_w3, a_b3,
              c_w0, c_b0, c_w1, c_b1, c_w2, c_b2, c_w3, c_b3,
              q1o_ref, q2o_ref):
    del be_ref
    valid = nv_ref[pl.program_id(0)]

    @pl.when(valid > 0)
    def _():
        rmask = lax.broadcasted_iota(jnp.int32, (BLK, QW), 0) < valid
        x = jnp.concatenate([xo_ref[...], xa_ref[...][:, :ACT_DIM]], axis=1)

        bf = jnp.bfloat16

        def head(w0, b0, w1, b1, w2, b2, w3, b3):
            h = jnp.dot(x.astype(bf), w0[0].astype(bf),
                        preferred_element_type=jnp.float32) + b0[0]
            h = jnp.maximum(h, 0.0)
            h = jnp.maximum(
                jnp.dot(h.astype(bf), w1[0].astype(bf),
                        preferred_element_type=jnp.float32) + b1[0], 0.0)
            h = jnp.maximum(
                jnp.dot(h.astype(bf), w2[0].astype(bf),
                        preferred_element_type=jnp.float32) + b2[0], 0.0)
            return jnp.dot(h.astype(bf), w3[0].astype(bf),
                           preferred_element_type=jnp.float32) + b3[0]

        q1 = head(a_w0, a_b0, a_w1, a_b1, a_w2, a_b2, a_w3, a_b3)
        q2 = head(c_w0, c_b0, c_w1, c_b1, c_w2, c_b2, c_w3, c_b3)
        q1o_ref[...] = jnp.where(rmask, jnp.broadcast_to(q1, (BLK, QW)), 0.0)
        q2o_ref[...] = jnp.where(rmask, jnp.broadcast_to(q2, (BLK, QW)), 0.0)


def _grouped_mlp(be, nv, xso, xsa, wts):
    def xmap(b, eb, nv_):
        return (b, 0)

    def emap3(b, eb, nv_):
        return (eb[b], 0, 0)

    w_specs = []
    for _ in range(2):
        w_specs += [
            pl.BlockSpec((1, OBS_DIM + ACT_DIM, HID), emap3),
            pl.BlockSpec((1, 1, HID), emap3),
            pl.BlockSpec((1, HID, HID), emap3),
            pl.BlockSpec((1, 1, HID), emap3),
            pl.BlockSpec((1, HID, HID), emap3),
            pl.BlockSpec((1, 1, HID), emap3),
            pl.BlockSpec((1, HID, 1), emap3),
            pl.BlockSpec((1, 1, 1), emap3),
        ]
    grid_spec = pltpu.PrefetchScalarGridSpec(
        num_scalar_prefetch=2,
        grid=(G,),
        in_specs=[pl.BlockSpec((BLK, OBS_DIM), xmap),
                  pl.BlockSpec((BLK, ACT_PAD), xmap)] + w_specs,
        out_specs=[pl.BlockSpec((BLK, QW), xmap),
                   pl.BlockSpec((BLK, QW), xmap)],
    )
    return pl.pallas_call(
        _mlp_body,
        grid_spec=grid_spec,
        out_shape=[jax.ShapeDtypeStruct((PAD, QW), jnp.float32),
                   jax.ShapeDtypeStruct((PAD, QW), jnp.float32)],
    )(be, nv, xso, xsa, *wts)


def _gather_body(q1p_hbm, q2p_hbm, pos_hbm, q1_hbm, q2_hbm, idx_v, buf, sem):
    wid = lax.axis_index("s") * NC + lax.axis_index("c")
    base = wid * RPW
    pltpu.sync_copy(pos_hbm.at[wid], idx_v)
    pltpu.async_copy(q1p_hbm.at[idx_v], buf, sem).wait()
    pltpu.sync_copy(buf, q1_hbm.at[pl.ds(base, RPW)])
    pltpu.async_copy(q2p_hbm.at[idx_v], buf, sem).wait()
    pltpu.sync_copy(buf, q2_hbm.at[pl.ds(base, RPW)])


def _gather(q1p, q2p, pos128):
    return pl.kernel(
        _gather_body,
        out_type=[jax.ShapeDtypeStruct((B, QW), jnp.float32),
                  jax.ShapeDtypeStruct((B, QW), jnp.float32)],
        mesh=plsc.VectorSubcoreMesh(core_axis_name="c", subcore_axis_name="s"),
        scratch_types=[
            pltpu.VMEM((RPW,), jnp.int32),
            pltpu.VMEM((RPW, QW), jnp.float32),
            pltpu.SemaphoreType.DMA,
        ],
    )(q1p, q2p, pos128)


def kernel(obs, actions,
           q1_W0, q1_b0, q1_W1, q1_b1, q1_W2, q1_b2, q1_W3, q1_b3,
           q2_W0, q2_b0, q2_W1, q2_b1, q2_W2, q2_b2, q2_W3, q2_b3):
    obs8 = obs[:, OBS_DIM - N_TASKS:].T
    pos2d, be2d, nv2d = _route(obs8)
    pos = pos2d.reshape(B)
    be = be2d.reshape(G)
    nv = nv2d.reshape(G)

    actions_pad = jnp.concatenate(
        [actions, jnp.zeros((B, ACT_PAD - ACT_DIM), jnp.float32)], axis=1)
    xso, xsa = _scatter(obs, actions_pad, pos.reshape(NW, NCH, CH))

    wts = []
    for (W0, b0, W1, b1, W2, b2, W3, b3) in (
            (q1_W0, q1_b0, q1_W1, q1_b1, q1_W2, q1_b2, q1_W3, q1_b3),
            (q2_W0, q2_b0, q2_W1, q2_b1, q2_W2, q2_b2, q2_W3, q2_b3)):
        wts += [W0, b0.reshape(N_TASKS, 1, HID),
                W1, b1.reshape(N_TASKS, 1, HID),
                W2, b2.reshape(N_TASKS, 1, HID),
                W3, b3.reshape(N_TASKS, 1, 1)]
    q1p, q2p = _grouped_mlp(be, nv, xso, xsa, wts)

    q1w, q2w = _gather(q1p, q2p, pos.reshape(NW, RPW))
    return q1w[:, :1], q2w[:, :1]


# speedup vs baseline: 3.5179x; 1.0182x over previous
import functools

import jax
import jax.numpy as jnp
from jax import lax
from jax.experimental import pallas as pl
from jax.experimental.pallas import tpu as pltpu
from jax.experimental.pallas import tpu_sc as plsc

B = 4096
OBS_DIM = 1024
ACT_DIM = 32
N_TASKS = 8
HID = 1024
BLK = 512
G = B // BLK + N_TASKS
PAD = G * BLK

ACT_PAD = 128
QW = 128
NC = 2
NS = 16
NW = NC * NS
RPW = B // NW
CH = 32
NCH = RPW // CH


def _route_body(obs8_ref, pos_ref, be_ref, nv_ref):
    v = obs8_ref[...]
    best = v[0:1]
    idx = jnp.zeros((1, B), jnp.int32)
    for t in range(1, N_TASKS):
        m = v[t:t + 1] > best
        idx = jnp.where(m, t, idx)
        best = jnp.where(m, v[t:t + 1], best)
    tid = lax.broadcasted_iota(jnp.int32, (N_TASKS, B), 0)
    oh = (idx == tid).astype(jnp.float32)
    cs = oh
    s = 1
    while s < B:
        cs = cs + jnp.concatenate(
            [jnp.zeros((N_TASKS, s), jnp.float32), cs[:, :B - s]], axis=1)
        s *= 2
    counts = cs[:, B - 1:B]
    nb = jnp.floor((counts + (BLK - 1)) / BLK)
    inc = nb
    s = 1
    while s < N_TASKS:
        inc = inc + jnp.concatenate(
            [jnp.zeros((s, 1), jnp.float32), inc[:N_TASKS - s]], axis=0)
        s *= 2
    base = (inc - nb) * BLK
    posf = jnp.sum(oh * (base + cs - 1.0), axis=0, keepdims=True)
    pos_ref[...] = posf.astype(jnp.int32)
    bi = lax.broadcasted_iota(jnp.int32, (N_TASKS, G), 1)
    inc_i = inc.astype(jnp.int32)
    nb_i = nb.astype(jnp.int32)
    be = jnp.sum((bi >= inc_i).astype(jnp.int32), axis=0, keepdims=True)
    be_ref[...] = jnp.minimum(be, N_TASKS - 1)
    first = inc_i - nb_i
    owner = jnp.logical_and(bi >= first, bi < inc_i)
    vraw = counts.astype(jnp.int32) - (bi - first) * BLK
    vclamp = jnp.clip(vraw, 0, BLK)
    nv_ref[...] = jnp.sum(jnp.where(owner, vclamp, 0), axis=0, keepdims=True)


def _route(obs8):
    return pl.pallas_call(
        _route_body,
        out_shape=[jax.ShapeDtypeStruct((1, B), jnp.int32),
                   jax.ShapeDtypeStruct((1, G), jnp.int32),
                   jax.ShapeDtypeStruct((1, G), jnp.int32)],
    )(obs8)


def _scatter_body(obs_hbm, act_hbm, pos_hbm, xso_hbm, xsa_hbm,
                  idx_v, ob0, ob1, ab0, ab1, lsem, ssem):
    wid = lax.axis_index("s") * NC + lax.axis_index("c")
    base = wid * RPW
    pltpu.sync_copy(pos_hbm.at[wid], idx_v)
    obufs, abufs = (ob0, ob1), (ab0, ab1)
    loads = {}

    def start_load(ch):
        loads[ch] = (
            pltpu.async_copy(obs_hbm.at[pl.ds(base + ch * CH, CH)],
                             obufs[ch % 2], lsem),
            pltpu.async_copy(act_hbm.at[pl.ds(base + ch * CH, CH)],
                             abufs[ch % 2], lsem),
        )

    start_load(0)
    prev_scat = None
    for ch in range(NCH):
        for h in loads.pop(ch):
            h.wait()
        if prev_scat is not None:
            for h in prev_scat:
                h.wait()
        if ch + 1 < NCH:
            start_load(ch + 1)
        prev_scat = (
            pltpu.async_copy(obufs[ch % 2], xso_hbm.at[idx_v.at[ch]], ssem),
            pltpu.async_copy(abufs[ch % 2], xsa_hbm.at[idx_v.at[ch]], ssem),
        )
    for h in prev_scat:
        h.wait()


def _scatter(obs, actions, pos3):
    return pl.kernel(
        _scatter_body,
        out_type=[jax.ShapeDtypeStruct((PAD, OBS_DIM), jnp.float32),
                  jax.ShapeDtypeStruct((PAD, ACT_PAD), jnp.float32)],
        mesh=plsc.VectorSubcoreMesh(core_axis_name="c", subcore_axis_name="s"),
        scratch_types=[
            pltpu.VMEM((NCH, CH), jnp.int32),
            pltpu.VMEM((CH, OBS_DIM), jnp.float32),
            pltpu.VMEM((CH, OBS_DIM), jnp.float32),
            pltpu.VMEM((CH, ACT_PAD), jnp.float32),
            pltpu.VMEM((CH, ACT_PAD), jnp.float32),
            pltpu.SemaphoreType.DMA,
            pltpu.SemaphoreType.DMA,
        ],
    )(obs, actions, pos3)


def _mlp_body(be_ref, nv_ref, xo_ref, xa_ref,
              a_w0, a_b0, a_w1, a_b1, a_w2, a_b2, a_w3, a_b3,
              c_w0, c_b0, c_w1, c_b1, c_w2, c_b2, c_w3, c_b3,
              q1o_ref, q2o_ref):
    del be_ref
    valid = nv_ref[pl.program_id(0)]

    @pl.when(valid > 0)
    def _():
        rmask = lax.broadcasted_iota(jnp.int32, (BLK, QW), 0) < valid
        x = jnp.concatenate([xo_ref[...], xa_ref[...][:, :ACT_DIM]], axis=1)

        bf = jnp.bfloat16

        def head(w0, b0, w1, b1, w2, b2, w3, b3):
            h = jnp.dot(x.astype(bf), w0[0].astype(bf),
                        preferred_element_type=jnp.float32) + b0[0]
            h = jnp.maximum(h, 0.0)
            h = jnp.maximum(
                jnp.dot(h.astype(bf), w1[0].astype(bf),
                        preferred_element_type=jnp.float32) + b1[0], 0.0)
            h = jnp.maximum(
                jnp.dot(h.astype(bf), w2[0].astype(bf),
                        preferred_element_type=jnp.float32) + b2[0], 0.0)
            return jnp.dot(h.astype(bf), w3[0].astype(bf),
                           preferred_element_type=jnp.float32) + b3[0]

        q1 = head(a_w0, a_b0, a_w1, a_b1, a_w2, a_b2, a_w3, a_b3)
        q2 = head(c_w0, c_b0, c_w1, c_b1, c_w2, c_b2, c_w3, c_b3)
        q1o_ref[...] = jnp.where(rmask, jnp.broadcast_to(q1, (BLK, QW)), 0.0)
        q2o_ref[...] = jnp.where(rmask, jnp.broadcast_to(q2, (BLK, QW)), 0.0)


def _grouped_mlp(be, nv, xso, xsa, wts):
    def xmap(b, eb, nv_):
        return (b, 0)

    def emap3(b, eb, nv_):
        return (eb[b], 0, 0)

    w_specs = []
    for _ in range(2):
        w_specs += [
            pl.BlockSpec((1, OBS_DIM + ACT_DIM, HID), emap3),
            pl.BlockSpec((1, 1, HID), emap3),
            pl.BlockSpec((1, HID, HID), emap3),
            pl.BlockSpec((1, 1, HID), emap3),
            pl.BlockSpec((1, HID, HID), emap3),
            pl.BlockSpec((1, 1, HID), emap3),
            pl.BlockSpec((1, HID, 1), emap3),
            pl.BlockSpec((1, 1, 1), emap3),
        ]
    grid_spec = pltpu.PrefetchScalarGridSpec(
        num_scalar_prefetch=2,
        grid=(G,),
        in_specs=[pl.BlockSpec((BLK, OBS_DIM), xmap),
                  pl.BlockSpec((BLK, ACT_PAD), xmap)] + w_specs,
        out_specs=[pl.BlockSpec((BLK, QW), xmap),
                   pl.BlockSpec((BLK, QW), xmap)],
    )
    return pl.pallas_call(
        _mlp_body,
        grid_spec=grid_spec,
        out_shape=[jax.ShapeDtypeStruct((PAD, QW), jnp.float32),
                   jax.ShapeDtypeStruct((PAD, QW), jnp.float32)],
    )(be, nv, xso, xsa, *wts)


def _gather_body(q1p_hbm, q2p_hbm, pos_hbm, q1_hbm, q2_hbm, idx_v, buf, sem):
    wid = lax.axis_index("s") * NC + lax.axis_index("c")
    base = wid * RPW
    pltpu.sync_copy(pos_hbm.at[wid], idx_v)
    pltpu.async_copy(q1p_hbm.at[idx_v], buf, sem).wait()
    pltpu.sync_copy(buf, q1_hbm.at[pl.ds(base, RPW)])
    pltpu.async_copy(q2p_hbm.at[idx_v], buf, sem).wait()
    pltpu.sync_copy(buf, q2_hbm.at[pl.ds(base, RPW)])


def _gather(q1p, q2p, pos128):
    return pl.kernel(
        _gather_body,
        out_type=[jax.ShapeDtypeStruct((B, QW), jnp.float32),
                  jax.ShapeDtypeStruct((B, QW), jnp.float32)],
        mesh=plsc.VectorSubcoreMesh(core_axis_name="c", subcore_axis_name="s"),
        scratch_types=[
            pltpu.VMEM((RPW,), jnp.int32),
            pltpu.VMEM((RPW, QW), jnp.float32),
            pltpu.SemaphoreType.DMA,
        ],
    )(q1p, q2p, pos128)


def kernel(obs, actions,
           q1_W0, q1_b0, q1_W1, q1_b1, q1_W2, q1_b2, q1_W3, q1_b3,
           q2_W0, q2_b0, q2_W1, q2_b1, q2_W2, q2_b2, q2_W3, q2_b3):
    obs8 = obs[:, OBS_DIM - N_TASKS:].T
    pos2d, be2d, nv2d = _route(obs8)
    pos = pos2d.reshape(B)
    be = be2d.reshape(G)
    nv = nv2d.reshape(G)

    actions_pad = jnp.concatenate(
        [actions, jnp.zeros((B, ACT_PAD - ACT_DIM), jnp.float32)], axis=1)
    xso, xsa = _scatter(obs, actions_pad, pos.reshape(NW, NCH, CH))

    wts = []
    for (W0, b0, W1, b1, W2, b2, W3, b3) in (
            (q1_W0, q1_b0, q1_W1, q1_b1, q1_W2, q1_b2, q1_W3, q1_b3),
            (q2_W0, q2_b0, q2_W1, q2_b1, q2_W2, q2_b2, q2_W3, q2_b3)):
        wts += [W0, b0.reshape(N_TASKS, 1, HID),
                W1, b1.reshape(N_TASKS, 1, HID),
                W2, b2.reshape(N_TASKS, 1, HID),
                W3, b3.reshape(N_TASKS, 1, 1)]
    q1p, q2p = _grouped_mlp(be, nv, xso, xsa, wts)

    q1w, q2w = _gather(q1p, q2p, pos.reshape(NW, RPW))
    return q1w[:, :1], q2w[:, :1]


# speedup vs baseline: 3.5399x; 1.0063x over previous
import functools

import jax
import jax.numpy as jnp
from jax import lax
from jax.experimental import pallas as pl
from jax.experimental.pallas import tpu as pltpu
from jax.experimental.pallas import tpu_sc as plsc

B = 4096
OBS_DIM = 1024
ACT_DIM = 32
N_TASKS = 8
HID = 1024
BLK = 512
G = B // BLK + N_TASKS
PAD = G * BLK

ACT_PAD = 128
QW = 128
NC = 2
NS = 16
NW = NC * NS
RPW = B // NW
CH = 32
NCH = RPW // CH


def _route_body(obs8_ref, pos_ref, be_ref, nv_ref,
                fi_ref, dp_ref, sl_ref, nx_ref):
    v = obs8_ref[...]
    best = v[0:1]
    idx = jnp.zeros((1, B), jnp.int32)
    for t in range(1, N_TASKS):
        m = v[t:t + 1] > best
        idx = jnp.where(m, t, idx)
        best = jnp.where(m, v[t:t + 1], best)
    tid = lax.broadcasted_iota(jnp.int32, (N_TASKS, B), 0)
    oh = (idx == tid).astype(jnp.float32)
    cs = oh
    s = 1
    while s < B:
        cs = cs + jnp.concatenate(
            [jnp.zeros((N_TASKS, s), jnp.float32), cs[:, :B - s]], axis=1)
        s *= 2
    counts = cs[:, B - 1:B]
    nb = jnp.floor((counts + (BLK - 1)) / BLK)
    inc = nb
    s = 1
    while s < N_TASKS:
        inc = inc + jnp.concatenate(
            [jnp.zeros((s, 1), jnp.float32), inc[:N_TASKS - s]], axis=0)
        s *= 2
    base = (inc - nb) * BLK
    posf = jnp.sum(oh * (base + cs - 1.0), axis=0, keepdims=True)
    pos_ref[...] = posf.astype(jnp.int32)
    bi = lax.broadcasted_iota(jnp.int32, (N_TASKS, G), 1)
    inc_i = inc.astype(jnp.int32)
    nb_i = nb.astype(jnp.int32)
    be = jnp.sum((bi >= inc_i).astype(jnp.int32), axis=0, keepdims=True)
    te = lax.broadcasted_iota(jnp.int32, (N_TASKS, 1), 0)
    last_e = jnp.max(jnp.where(nb_i > 0, te, -1), axis=0, keepdims=True)
    be = jnp.minimum(be, last_e)
    be_ref[...] = be
    first = inc_i - nb_i
    owner = jnp.logical_and(bi >= first, bi < inc_i)
    vraw = counts.astype(jnp.int32) - (bi - first) * BLK
    vclamp = jnp.clip(vraw, 0, BLK)
    nv_ref[...] = jnp.sum(jnp.where(owner, vclamp, 0), axis=0, keepdims=True)
    one = jnp.ones((1, 1), jnp.int32)
    chg = jnp.concatenate([one, (be[:, 1:] != be[:, :-1]).astype(jnp.int32)],
                          axis=1)
    grp = chg
    s = 1
    while s < G:
        grp = grp + jnp.concatenate(
            [jnp.zeros((1, s), jnp.int32), grp[:, :G - s]], axis=1)
        s *= 2
    sl_ref[...] = (grp - 1) % 2
    fi_ref[...] = chg
    nxt = be
    found = jnp.zeros((1, G), jnp.int32)
    lastv = be[:, G - 1:G]
    for k in range(1, G):
        sh = jnp.concatenate(
            [be[:, k:], jnp.broadcast_to(lastv, (1, k))], axis=1)
        take = jnp.logical_and(found == 0, sh != be)
        nxt = jnp.where(take, sh, nxt)
        found = jnp.where(take, 1, found)
    nx_ref[...] = nxt
    dp_ref[...] = chg * found


def _route(obs8):
    return pl.pallas_call(
        _route_body,
        out_shape=[jax.ShapeDtypeStruct((1, B), jnp.int32)]
        + [jax.ShapeDtypeStruct((1, G), jnp.int32)] * 6,
    )(obs8)


def _scatter_body(obs_hbm, act_hbm, pos_hbm, xso_hbm, xsa_hbm,
                  idx_v, ob0, ob1, ab0, ab1, lsem, ssem):
    wid = lax.axis_index("s") * NC + lax.axis_index("c")
    base = wid * RPW
    pltpu.sync_copy(pos_hbm.at[wid], idx_v)
    obufs, abufs = (ob0, ob1), (ab0, ab1)
    loads = {}

    def start_load(ch):
        loads[ch] = (
            pltpu.async_copy(obs_hbm.at[pl.ds(base + ch * CH, CH)],
                             obufs[ch % 2], lsem),
            pltpu.async_copy(act_hbm.at[pl.ds(base + ch * CH, CH)],
                             abufs[ch % 2], lsem),
        )

    start_load(0)
    prev_scat = None
    for ch in range(NCH):
        for h in loads.pop(ch):
            h.wait()
        if prev_scat is not None:
            for h in prev_scat:
                h.wait()
        if ch + 1 < NCH:
            start_load(ch + 1)
        prev_scat = (
            pltpu.async_copy(obufs[ch % 2], xso_hbm.at[idx_v.at[ch]], ssem),
            pltpu.async_copy(abufs[ch % 2], xsa_hbm.at[idx_v.at[ch]], ssem),
        )
    for h in prev_scat:
        h.wait()


def _scatter(obs, actions, pos3):
    return pl.kernel(
        _scatter_body,
        out_type=[jax.ShapeDtypeStruct((PAD, OBS_DIM), jnp.float32),
                  jax.ShapeDtypeStruct((PAD, ACT_PAD), jnp.float32)],
        mesh=plsc.VectorSubcoreMesh(core_axis_name="c", subcore_axis_name="s"),
        scratch_types=[
            pltpu.VMEM((NCH, CH), jnp.int32),
            pltpu.VMEM((CH, OBS_DIM), jnp.float32),
            pltpu.VMEM((CH, OBS_DIM), jnp.float32),
            pltpu.VMEM((CH, ACT_PAD), jnp.float32),
            pltpu.VMEM((CH, ACT_PAD), jnp.float32),
            pltpu.SemaphoreType.DMA,
            pltpu.SemaphoreType.DMA,
        ],
    )(obs, actions, pos3)


def _mlp_body(be_s, nv_s, fi_s, dp_s, sl_s, nx_s, xo_ref, xa_ref,
              a_w0, a_b0, a_w1, a_b1, a_w2, a_b2, a_w3, a_b3,
              c_w0, c_b0, c_w1, c_b1, c_w2, c_b2, c_w3, c_b3,
              q1o_ref, q2o_ref,
              w0s_a, b0s_a, w1s_a, b1s_a, w2s_a, b2s_a, w3s_a, b3s_a,
              w0s_c, b0s_c, w1s_c, b1s_c, w2s_c, b2s_c, w3s_c, b3s_c,
              sems):
    b = pl.program_id(0)
    sl = sl_s[b]
    pairs = ((a_w0, w0s_a), (a_b0, b0s_a), (a_w1, w1s_a), (a_b1, b1s_a),
             (a_w2, w2s_a), (a_b2, b2s_a), (a_w3, w3s_a), (a_b3, b3s_a),
             (c_w0, w0s_c), (c_b0, b0s_c), (c_w1, w1s_c), (c_b1, b1s_c),
             (c_w2, w2s_c), (c_b2, b2s_c), (c_w3, w3s_c), (c_b3, b3s_c))

    def fetch(e, s):
        for hbm, buf in pairs:
            pltpu.make_async_copy(hbm.at[pl.ds(e, 1)], buf.at[pl.ds(s, 1)],
                                  sems.at[s]).start()

    def wait(s):
        for hbm, buf in pairs:
            pltpu.make_async_copy(hbm.at[pl.ds(0, 1)], buf.at[pl.ds(s, 1)],
                                  sems.at[s]).wait()

    @pl.when(b == 0)
    def _():
        fetch(be_s[0], 0)

    @pl.when(fi_s[b] == 1)
    def _():
        @pl.when(dp_s[b] == 1)
        def _():
            fetch(nx_s[b], 1 - sl)
        wait(sl)

    valid = nv_s[b]

    @pl.when(valid > 0)
    def _():
        rmask = lax.broadcasted_iota(jnp.int32, (BLK, QW), 0) < valid
        x = jnp.concatenate([xo_ref[...], xa_ref[...][:, :ACT_DIM]], axis=1)

        bf = jnp.bfloat16
        ds = pl.ds(sl, 1)

        def head(w0s, b0s, w1s, b1s, w2s, b2s, w3s, b3s):
            h = jnp.dot(x.astype(bf), w0s[ds][0].astype(bf),
                        preferred_element_type=jnp.float32) + b0s[ds][0]
            h = jnp.maximum(h, 0.0)
            h = jnp.maximum(
                jnp.dot(h.astype(bf), w1s[ds][0].astype(bf),
                        preferred_element_type=jnp.float32) + b1s[ds][0], 0.0)
            h = jnp.maximum(
                jnp.dot(h.astype(bf), w2s[ds][0].astype(bf),
                        preferred_element_type=jnp.float32) + b2s[ds][0], 0.0)
            return jnp.dot(h.astype(bf), w3s[ds][0].astype(bf),
                           preferred_element_type=jnp.float32) + b3s[ds][0]

        q1 = head(w0s_a, b0s_a, w1s_a, b1s_a, w2s_a, b2s_a, w3s_a, b3s_a)
        q2 = head(w0s_c, b0s_c, w1s_c, b1s_c, w2s_c, b2s_c, w3s_c, b3s_c)
        q1o_ref[...] = jnp.where(rmask, jnp.broadcast_to(q1, (BLK, QW)), 0.0)
        q2o_ref[...] = jnp.where(rmask, jnp.broadcast_to(q2, (BLK, QW)), 0.0)


def _grouped_mlp(be, nv, fi, dp, sl, nx, xso, xsa, wts):
    def xmap(b, *_):
        return (b, 0)

    w_specs = [pl.BlockSpec(memory_space=pl.ANY)] * 16
    wring = []
    for _ in range(2):
        wring += [
            pltpu.VMEM((2, OBS_DIM + ACT_DIM, HID), jnp.float32),
            pltpu.VMEM((2, 1, HID), jnp.float32),
            pltpu.VMEM((2, HID, HID), jnp.float32),
            pltpu.VMEM((2, 1, HID), jnp.float32),
            pltpu.VMEM((2, HID, HID), jnp.float32),
            pltpu.VMEM((2, 1, HID), jnp.float32),
            pltpu.VMEM((2, HID, 1), jnp.float32),
            pltpu.VMEM((2, 1, 1), jnp.float32),
        ]
    grid_spec = pltpu.PrefetchScalarGridSpec(
        num_scalar_prefetch=6,
        grid=(G,),
        in_specs=[pl.BlockSpec((BLK, OBS_DIM), xmap),
                  pl.BlockSpec((BLK, ACT_PAD), xmap)] + w_specs,
        out_specs=[pl.BlockSpec((BLK, QW), xmap),
                   pl.BlockSpec((BLK, QW), xmap)],
        scratch_shapes=wring + [pltpu.SemaphoreType.DMA((2,))],
    )
    return pl.pallas_call(
        _mlp_body,
        grid_spec=grid_spec,
        out_shape=[jax.ShapeDtypeStruct((PAD, QW), jnp.float32),
                   jax.ShapeDtypeStruct((PAD, QW), jnp.float32)],
    )(be, nv, fi, dp, sl, nx, xso, xsa, *wts)


def _gather_body(q1p_hbm, q2p_hbm, pos_hbm, q1_hbm, q2_hbm, idx_v, buf, sem):
    wid = lax.axis_index("s") * NC + lax.axis_index("c")
    base = wid * RPW
    pltpu.sync_copy(pos_hbm.at[wid], idx_v)
    pltpu.async_copy(q1p_hbm.at[idx_v], buf, sem).wait()
    pltpu.sync_copy(buf, q1_hbm.at[pl.ds(base, RPW)])
    pltpu.async_copy(q2p_hbm.at[idx_v], buf, sem).wait()
    pltpu.sync_copy(buf, q2_hbm.at[pl.ds(base, RPW)])


def _gather(q1p, q2p, pos128):
    return pl.kernel(
        _gather_body,
        out_type=[jax.ShapeDtypeStruct((B, QW), jnp.float32),
                  jax.ShapeDtypeStruct((B, QW), jnp.float32)],
        mesh=plsc.VectorSubcoreMesh(core_axis_name="c", subcore_axis_name="s"),
        scratch_types=[
            pltpu.VMEM((RPW,), jnp.int32),
            pltpu.VMEM((RPW, QW), jnp.float32),
            pltpu.SemaphoreType.DMA,
        ],
    )(q1p, q2p, pos128)


def kernel(obs, actions,
           q1_W0, q1_b0, q1_W1, q1_b1, q1_W2, q1_b2, q1_W3, q1_b3,
           q2_W0, q2_b0, q2_W1, q2_b1, q2_W2, q2_b2, q2_W3, q2_b3):
    obs8 = obs[:, OBS_DIM - N_TASKS:].T
    pos2d, be2d, nv2d, fi2d, dp2d, sl2d, nx2d = _route(obs8)
    pos = pos2d.reshape(B)
    be, nv, fi, dp, sl, nx = (a.reshape(G) for a in
                              (be2d, nv2d, fi2d, dp2d, sl2d, nx2d))

    actions_pad = jnp.concatenate(
        [actions, jnp.zeros((B, ACT_PAD - ACT_DIM), jnp.float32)], axis=1)
    xso, xsa = _scatter(obs, actions_pad, pos.reshape(NW, NCH, CH))

    wts = []
    for (W0, b0, W1, b1, W2, b2, W3, b3) in (
            (q1_W0, q1_b0, q1_W1, q1_b1, q1_W2, q1_b2, q1_W3, q1_b3),
            (q2_W0, q2_b0, q2_W1, q2_b1, q2_W2, q2_b2, q2_W3, q2_b3)):
        wts += [W0, b0.reshape(N_TASKS, 1, HID),
                W1, b1.reshape(N_TASKS, 1, HID),
                W2, b2.reshape(N_TASKS, 1, HID),
                W3, b3.reshape(N_TASKS, 1, 1)]
    q1p, q2p = _grouped_mlp(be, nv, fi, dp, sl, nx, xso, xsa, wts)

    q1w, q2w = _gather(q1p, q2p, pos.reshape(NW, RPW))
    return q1w[:, :1], q2w[:, :1]


# speedup vs baseline: 3.6145x; 1.0211x over previous
import functools

import jax
import jax.numpy as jnp
from jax import lax
from jax.experimental import pallas as pl
from jax.experimental.pallas import tpu as pltpu
from jax.experimental.pallas import tpu_sc as plsc

B = 4096
OBS_DIM = 1024
ACT_DIM = 32
N_TASKS = 8
HID = 1024
BLK = 512
G = B // BLK + N_TASKS
PAD = G * BLK

ACT_PAD = 128
QW = 128
NC = 2
NS = 16
NW = NC * NS
RPW = B // NW
CH = 32
NCH = RPW // CH


def _route_body(obs8_ref, pos_ref, be_ref, nv_ref,
                fi_ref, dp_ref, sl_ref, nx_ref):
    v = obs8_ref[...]
    best = v[0:1]
    idx = jnp.zeros((1, B), jnp.int32)
    for t in range(1, N_TASKS):
        m = v[t:t + 1] > best
        idx = jnp.where(m, t, idx)
        best = jnp.where(m, v[t:t + 1], best)
    tid = lax.broadcasted_iota(jnp.int32, (N_TASKS, B), 0)
    oh = (idx == tid).astype(jnp.float32)
    cs = oh
    s = 1
    while s < B:
        cs = cs + jnp.concatenate(
            [jnp.zeros((N_TASKS, s), jnp.float32), cs[:, :B - s]], axis=1)
        s *= 2
    counts = cs[:, B - 1:B]
    nb = jnp.floor((counts + (BLK - 1)) / BLK)
    inc = nb
    s = 1
    while s < N_TASKS:
        inc = inc + jnp.concatenate(
            [jnp.zeros((s, 1), jnp.float32), inc[:N_TASKS - s]], axis=0)
        s *= 2
    base = (inc - nb) * BLK
    posf = jnp.sum(oh * (base + cs - 1.0), axis=0, keepdims=True)
    pos_ref[...] = posf.astype(jnp.int32)
    bi = lax.broadcasted_iota(jnp.int32, (N_TASKS, G), 1)
    inc_i = inc.astype(jnp.int32)
    nb_i = nb.astype(jnp.int32)
    be = jnp.sum((bi >= inc_i).astype(jnp.int32), axis=0, keepdims=True)
    te = lax.broadcasted_iota(jnp.int32, (N_TASKS, 1), 0)
    last_e = jnp.max(jnp.where(nb_i > 0, te, -1), axis=0, keepdims=True)
    be = jnp.minimum(be, last_e)
    be_ref[...] = be
    first = inc_i - nb_i
    owner = jnp.logical_and(bi >= first, bi < inc_i)
    vraw = counts.astype(jnp.int32) - (bi - first) * BLK
    vclamp = jnp.clip(vraw, 0, BLK)
    nv_ref[...] = jnp.sum(jnp.where(owner, vclamp, 0), axis=0, keepdims=True)
    one = jnp.ones((1, 1), jnp.int32)
    chg = jnp.concatenate([one, (be[:, 1:] != be[:, :-1]).astype(jnp.int32)],
                          axis=1)
    grp = chg
    s = 1
    while s < G:
        grp = grp + jnp.concatenate(
            [jnp.zeros((1, s), jnp.int32), grp[:, :G - s]], axis=1)
        s *= 2
    sl_ref[...] = (grp - 1) % 2
    fi_ref[...] = chg
    nxt = be
    found = jnp.zeros((1, G), jnp.int32)
    lastv = be[:, G - 1:G]
    for k in range(1, G):
        sh = jnp.concatenate(
            [be[:, k:], jnp.broadcast_to(lastv, (1, k))], axis=1)
        take = jnp.logical_and(found == 0, sh != be)
        nxt = jnp.where(take, sh, nxt)
        found = jnp.where(take, 1, found)
    nx_ref[...] = nxt
    dp_ref[...] = chg * found


def _route(obs8):
    return pl.pallas_call(
        _route_body,
        out_shape=[jax.ShapeDtypeStruct((1, B), jnp.int32)]
        + [jax.ShapeDtypeStruct((1, G), jnp.int32)] * 6,
    )(obs8)


def _scatter_body(obs_hbm, act_hbm, pos_hbm, xso_hbm, xsa_hbm,
                  idx_v, ob0, ob1, ab0, ab1, lsem, ssem):
    wid = lax.axis_index("s") * NC + lax.axis_index("c")
    base = wid * RPW
    pltpu.sync_copy(pos_hbm.at[wid], idx_v)
    obufs, abufs = (ob0, ob1), (ab0, ab1)
    loads = {}

    def start_load(ch):
        loads[ch] = (
            pltpu.async_copy(obs_hbm.at[pl.ds(base + ch * CH, CH)],
                             obufs[ch % 2], lsem),
            pltpu.async_copy(act_hbm.at[pl.ds(base + ch * CH, CH)],
                             abufs[ch % 2], lsem),
        )

    start_load(0)
    prev_scat = None
    for ch in range(NCH):
        for h in loads.pop(ch):
            h.wait()
        if prev_scat is not None:
            for h in prev_scat:
                h.wait()
        if ch + 1 < NCH:
            start_load(ch + 1)
        prev_scat = (
            pltpu.async_copy(obufs[ch % 2], xso_hbm.at[idx_v.at[ch]], ssem),
            pltpu.async_copy(abufs[ch % 2], xsa_hbm.at[idx_v.at[ch]], ssem),
        )
    for h in prev_scat:
        h.wait()


def _scatter(obs, actions, pos3):
    return pl.kernel(
        _scatter_body,
        out_type=[jax.ShapeDtypeStruct((PAD, OBS_DIM), jnp.float32),
                  jax.ShapeDtypeStruct((PAD, ACT_PAD), jnp.float32)],
        mesh=plsc.VectorSubcoreMesh(core_axis_name="c", subcore_axis_name="s"),
        scratch_types=[
            pltpu.VMEM((NCH, CH), jnp.int32),
            pltpu.VMEM((CH, OBS_DIM), jnp.float32),
            pltpu.VMEM((CH, OBS_DIM), jnp.float32),
            pltpu.VMEM((CH, ACT_PAD), jnp.float32),
            pltpu.VMEM((CH, ACT_PAD), jnp.float32),
            pltpu.SemaphoreType.DMA,
            pltpu.SemaphoreType.DMA,
        ],
    )(obs, actions, pos3)


def _mlp_body(be_s, nv_s, fi_s, dp_s, sl_s, nx_s, xo_ref, xa_ref,
              a_w0, a_b0, a_w1, a_b1, a_w2, a_b2, a_w3, a_b3,
              c_w0, c_b0, c_w1, c_b1, c_w2, c_b2, c_w3, c_b3,
              qo_ref,
              w0s_a, b0s_a, w1s_a, b1s_a, w2s_a, b2s_a, w3s_a, b3s_a,
              w0s_c, b0s_c, w1s_c, b1s_c, w2s_c, b2s_c, w3s_c, b3s_c,
              sems):
    b = pl.program_id(0)
    sl = sl_s[b]
    pairs = ((a_w0, w0s_a), (a_b0, b0s_a), (a_w1, w1s_a), (a_b1, b1s_a),
             (a_w2, w2s_a), (a_b2, b2s_a), (a_w3, w3s_a), (a_b3, b3s_a),
             (c_w0, w0s_c), (c_b0, b0s_c), (c_w1, w1s_c), (c_b1, b1s_c),
             (c_w2, w2s_c), (c_b2, b2s_c), (c_w3, w3s_c), (c_b3, b3s_c))

    def fetch(e, s):
        for hbm, buf in pairs:
            pltpu.make_async_copy(hbm.at[pl.ds(e, 1)], buf.at[pl.ds(s, 1)],
                                  sems.at[s]).start()

    def wait(s):
        for hbm, buf in pairs:
            pltpu.make_async_copy(hbm.at[pl.ds(0, 1)], buf.at[pl.ds(s, 1)],
                                  sems.at[s]).wait()

    @pl.when(b == 0)
    def _():
        fetch(be_s[0], 0)

    @pl.when(fi_s[b] == 1)
    def _():
        @pl.when(dp_s[b] == 1)
        def _():
            fetch(nx_s[b], 1 - sl)
        wait(sl)

    valid = nv_s[b]

    @pl.when(valid > 0)
    def _():
        rmask = lax.broadcasted_iota(jnp.int32, (BLK, QW), 0) < valid
        x = jnp.concatenate([xo_ref[...], xa_ref[...][:, :ACT_DIM]], axis=1)

        bf = jnp.bfloat16
        ds = pl.ds(sl, 1)

        def head(w0s, b0s, w1s, b1s, w2s, b2s, w3s, b3s):
            h = jnp.dot(x.astype(bf), w0s[ds][0].astype(bf),
                        preferred_element_type=jnp.float32) + b0s[ds][0]
            h = jnp.maximum(h, 0.0)
            h = jnp.maximum(
                jnp.dot(h.astype(bf), w1s[ds][0].astype(bf),
                        preferred_element_type=jnp.float32) + b1s[ds][0], 0.0)
            h = jnp.maximum(
                jnp.dot(h.astype(bf), w2s[ds][0].astype(bf),
                        preferred_element_type=jnp.float32) + b2s[ds][0], 0.0)
            return jnp.dot(h.astype(bf), w3s[ds][0].astype(bf),
                           preferred_element_type=jnp.float32) + b3s[ds][0]

        q1 = head(w0s_a, b0s_a, w1s_a, b1s_a, w2s_a, b2s_a, w3s_a, b3s_a)
        q2 = head(w0s_c, b0s_c, w1s_c, b1s_c, w2s_c, b2s_c, w3s_c, b3s_c)
        qq = jnp.concatenate([jnp.broadcast_to(q1, (BLK, QW // 2)),
                              jnp.broadcast_to(q2, (BLK, QW // 2))], axis=1)
        qo_ref[...] = jnp.where(rmask, qq, 0.0)


def _grouped_mlp(be, nv, fi, dp, sl, nx, xso, xsa, wts):
    def xmap(b, *_):
        return (b, 0)

    w_specs = [pl.BlockSpec(memory_space=pl.ANY)] * 16
    wring = []
    for _ in range(2):
        wring += [
            pltpu.VMEM((2, OBS_DIM + ACT_DIM, HID), jnp.float32),
            pltpu.VMEM((2, 1, HID), jnp.float32),
            pltpu.VMEM((2, HID, HID), jnp.float32),
            pltpu.VMEM((2, 1, HID), jnp.float32),
            pltpu.VMEM((2, HID, HID), jnp.float32),
            pltpu.VMEM((2, 1, HID), jnp.float32),
            pltpu.VMEM((2, HID, 1), jnp.float32),
            pltpu.VMEM((2, 1, 1), jnp.float32),
        ]
    grid_spec = pltpu.PrefetchScalarGridSpec(
        num_scalar_prefetch=6,
        grid=(G,),
        in_specs=[pl.BlockSpec((BLK, OBS_DIM), xmap),
                  pl.BlockSpec((BLK, ACT_PAD), xmap)] + w_specs,
        out_specs=pl.BlockSpec((BLK, QW), xmap),
        scratch_shapes=wring + [pltpu.SemaphoreType.DMA((2,))],
    )
    return pl.pallas_call(
        _mlp_body,
        grid_spec=grid_spec,
        out_shape=jax.ShapeDtypeStruct((PAD, QW), jnp.float32),
    )(be, nv, fi, dp, sl, nx, xso, xsa, *wts)


def _gather_body(qp_hbm, pos_hbm, q_hbm, idx_v, buf, sem):
    wid = lax.axis_index("s") * NC + lax.axis_index("c")
    base = wid * RPW
    pltpu.sync_copy(pos_hbm.at[wid], idx_v)
    pltpu.async_copy(qp_hbm.at[idx_v], buf, sem).wait()
    pltpu.sync_copy(buf, q_hbm.at[pl.ds(base, RPW)])


def _gather(qp, pos128):
    return pl.kernel(
        _gather_body,
        out_type=jax.ShapeDtypeStruct((B, QW), jnp.float32),
        mesh=plsc.VectorSubcoreMesh(core_axis_name="c", subcore_axis_name="s"),
        scratch_types=[
            pltpu.VMEM((RPW,), jnp.int32),
            pltpu.VMEM((RPW, QW), jnp.float32),
            pltpu.SemaphoreType.DMA,
        ],
    )(qp, pos128)


def kernel(obs, actions,
           q1_W0, q1_b0, q1_W1, q1_b1, q1_W2, q1_b2, q1_W3, q1_b3,
           q2_W0, q2_b0, q2_W1, q2_b1, q2_W2, q2_b2, q2_W3, q2_b3):
    obs8 = obs[:, OBS_DIM - N_TASKS:].T
    pos2d, be2d, nv2d, fi2d, dp2d, sl2d, nx2d = _route(obs8)
    pos = pos2d.reshape(B)
    be, nv, fi, dp, sl, nx = (a.reshape(G) for a in
                              (be2d, nv2d, fi2d, dp2d, sl2d, nx2d))

    actions_pad = jnp.concatenate(
        [actions, jnp.zeros((B, ACT_PAD - ACT_DIM), jnp.float32)], axis=1)
    xso, xsa = _scatter(obs, actions_pad, pos.reshape(NW, NCH, CH))

    wts = []
    for (W0, b0, W1, b1, W2, b2, W3, b3) in (
            (q1_W0, q1_b0, q1_W1, q1_b1, q1_W2, q1_b2, q1_W3, q1_b3),
            (q2_W0, q2_b0, q2_W1, q2_b1, q2_W2, q2_b2, q2_W3, q2_b3)):
        wts += [W0, b0.reshape(N_TASKS, 1, HID),
                W1, b1.reshape(N_TASKS, 1, HID),
                W2, b2.reshape(N_TASKS, 1, HID),
                W3, b3.reshape(N_TASKS, 1, 1)]
    qp = _grouped_mlp(be, nv, fi, dp, sl, nx, xso, xsa, wts)

    qw = _gather(qp, pos.reshape(NW, RPW))
    return qw[:, :1], qw[:, QW // 2:QW // 2 + 1]
